# baseline (speedup 1.0000x reference)
import functools
import jax
import jax.numpy as jnp
from jax.experimental import pallas as pl

N = 10000
T = 256
HID = 512
L = 256

BR = 1000


def _mm_stats_body(x_ref, agg_ref, w_ref, b_ref, h_ref, s1_ref, s2_ref):
    i = pl.program_id(0)
    u = x_ref[...] + agg_ref[...]
    h = jnp.dot(u, w_ref[...], preferred_element_type=jnp.float32) + b_ref[...]
    h_ref[...] = h

    @pl.when(i == 0)
    def _():
        s1_ref[...] = jnp.zeros_like(s1_ref)
        s2_ref[...] = jnp.zeros_like(s2_ref)

    s1_ref[...] += jnp.sum(h, axis=0, keepdims=True)
    s2_ref[...] += jnp.sum(h * h, axis=0, keepdims=True)


def _mm_stats(x, agg, w, b):
    n, k = x.shape
    c = w.shape[1]
    return pl.pallas_call(
        _mm_stats_body,
        grid=(n // BR,),
        in_specs=[
            pl.BlockSpec((BR, k), lambda i: (i, 0)),
            pl.BlockSpec((BR, k), lambda i: (i, 0)),
            pl.BlockSpec((k, c), lambda i: (0, 0)),
            pl.BlockSpec((1, c), lambda i: (0, 0)),
        ],
        out_specs=[
            pl.BlockSpec((BR, c), lambda i: (i, 0)),
            pl.BlockSpec((1, c), lambda i: (0, 0)),
            pl.BlockSpec((1, c), lambda i: (0, 0)),
        ],
        out_shape=[
            jax.ShapeDtypeStruct((n, c), jnp.float32),
            jax.ShapeDtypeStruct((1, c), jnp.float32),
            jax.ShapeDtypeStruct((1, c), jnp.float32),
        ],
    )(x, agg, w, b.reshape(1, c))


def _stats_body(s1_ref, s2_ref, g_ref, be_ref, a_ref, sh_ref):
    mean = s1_ref[...] * (1.0 / N)
    var = s2_ref[...] * (1.0 / N) - mean * mean
    a = g_ref[...] * jax.lax.rsqrt(var + 1e-5)
    a_ref[...] = a
    sh_ref[...] = be_ref[...] - mean * a


def _bn_scale(s1, s2, g, be):
    c = s1.shape[1]
    return pl.pallas_call(
        _stats_body,
        out_shape=[jax.ShapeDtypeStruct((1, c), jnp.float32),
                   jax.ShapeDtypeStruct((1, c), jnp.float32)],
    )(s1, s2, g.reshape(1, c), be.reshape(1, c))


def _bn_mm_body(h_ref, a_ref, sh_ref, w_ref, b_ref, o_ref, *, relu_out):
    t = jnp.maximum(h_ref[...] * a_ref[...] + sh_ref[...], 0.0)
    o = jnp.dot(t, w_ref[...], preferred_element_type=jnp.float32) + b_ref[...]
    if relu_out:
        o = jnp.maximum(o, 0.0)
    o_ref[...] = o


def _bn_mm(h, a, sh, w, b, relu_out):
    n, k = h.shape
    c = w.shape[1]
    return pl.pallas_call(
        functools.partial(_bn_mm_body, relu_out=relu_out),
        grid=(n // BR,),
        in_specs=[
            pl.BlockSpec((BR, k), lambda i: (i, 0)),
            pl.BlockSpec((1, k), lambda i: (0, 0)),
            pl.BlockSpec((1, k), lambda i: (0, 0)),
            pl.BlockSpec((k, c), lambda i: (0, 0)),
            pl.BlockSpec((1, c), lambda i: (0, 0)),
        ],
        out_specs=pl.BlockSpec((BR, c), lambda i: (i, 0)),
        out_shape=jax.ShapeDtypeStruct((n, c), jnp.float32),
    )(h, a, sh, w, b.reshape(1, c))


def _dual_mm_body(x_ref, wl_ref, bl_ref, wr_ref, br_ref, l_ref, r_ref):
    x = x_ref[...]
    l_ref[...] = jnp.dot(x, wl_ref[...], preferred_element_type=jnp.float32) + bl_ref[...]
    r_ref[...] = jnp.dot(x, wr_ref[...], preferred_element_type=jnp.float32) + br_ref[...]


def _dual_mm(x, wl, bl, wr, br):
    n, k = x.shape
    c = wl.shape[1]
    return pl.pallas_call(
        _dual_mm_body,
        grid=(n // BR,),
        in_specs=[
            pl.BlockSpec((BR, k), lambda i: (i, 0)),
            pl.BlockSpec((k, c), lambda i: (0, 0)),
            pl.BlockSpec((1, c), lambda i: (0, 0)),
            pl.BlockSpec((k, c), lambda i: (0, 0)),
            pl.BlockSpec((1, c), lambda i: (0, 0)),
        ],
        out_specs=[pl.BlockSpec((BR, c), lambda i: (i, 0)),
                   pl.BlockSpec((BR, c), lambda i: (i, 0))],
        out_shape=[jax.ShapeDtypeStruct((n, c), jnp.float32),
                   jax.ShapeDtypeStruct((n, c), jnp.float32)],
    )(x, wl, bl.reshape(1, c), wr, br.reshape(1, c))


def _final_body(flat_ref, wd_ref, bd_ref, out_ref):
    i = pl.program_id(0)

    @pl.when(i == 0)
    def _():
        out_ref[...] = jnp.zeros_like(out_ref)

    out_ref[...] += jnp.sum(flat_ref[...] * wd_ref[...]).reshape(1, 1)

    @pl.when(i == pl.num_programs(0) - 1)
    def _():
        out_ref[...] = jax.nn.sigmoid(out_ref[...] + bd_ref[...])


def _final_dot(flat, wd, bd):
    return pl.pallas_call(
        _final_body,
        grid=(N // BR,),
        in_specs=[pl.BlockSpec((BR, L), lambda i: (i, 0)),
                  pl.BlockSpec((BR, L), lambda i: (i, 0)),
                  pl.BlockSpec((1, 1), lambda i: (0, 0))],
        out_specs=pl.BlockSpec((1, 1), lambda i: (0, 0)),
        out_shape=jax.ShapeDtypeStruct((1, 1), jnp.float32),
    )(flat, wd, bd.reshape(1, 1))


def _gin_conv(x, src, dst, W1, b1, g, be, W2, b2, relu_out):
    agg = jax.ops.segment_sum(x[src], dst, num_segments=N)
    h, s1, s2 = _mm_stats(x, agg, W1, b1)
    a, sh = _bn_scale(s1, s2, g, be)
    return _bn_mm(h, a, sh, W2, b2, relu_out)


def _gatv2_edges(xl, xr, s, d, att, heads, out_ch):
    e = jax.nn.leaky_relu(xl[s] + xr[d], 0.2)
    alpha = (e * att).sum(-1)
    m = jax.ops.segment_max(alpha, d, num_segments=N)
    alpha = jnp.exp(alpha - m[d])
    denom = jax.ops.segment_sum(alpha, d, num_segments=N)
    alpha = alpha / (denom[d] + 1e-16)
    return jax.ops.segment_sum(xl[s] * alpha[:, :, None], d, num_segments=N)


def kernel(eeg_nodes, eeg_idx, W11, b11, g1, be1, W12, b12, W21, b21, g2, be2, W22, b22,
           Wl1, bl1, Wr1, br1, att1, bias1, Wl2, bl2, Wr2, br2, att2, bias2, Wd, bd):
    src = eeg_idx[0]
    dst = eeg_idx[1]
    loop = jnp.arange(N, dtype=src.dtype)
    s2_ = jnp.concatenate([src, loop])
    d2_ = jnp.concatenate([dst, loop])

    h = _gin_conv(eeg_nodes, src, dst, W11, b11, g1, be1, W12, b12, relu_out=True)
    h = _gin_conv(h, src, dst, W21, b21, g2, be2, W22, b22, relu_out=False)

    xl1, xr1 = _dual_mm(h, Wl1, bl1, Wr1, br1)
    xl1 = xl1.reshape(N, 4, L)
    xr1 = xr1.reshape(N, 4, L)
    r1 = _gatv2_edges(xl1, xr1, s2_, d2_, att1, 4, L).reshape(N, 4 * L) + bias1

    xl2 = (r1 @ Wl2 + bl2).reshape(N, 1, 1)
    xr2 = (r1 @ Wr2 + br2).reshape(N, 1, 1)
    region_scores = _gatv2_edges(xl2, xr2, s2_, d2_, att2, 1, 1).reshape(N, 1) + bias2

    dementia_pred = _final_dot(h, Wd.reshape(N, L), bd)
    return (dementia_pred, region_scores)


# speedup vs baseline: 1.0724x; 1.0724x over previous
import functools
import jax
import jax.numpy as jnp
from jax import lax
from jax.experimental import pallas as pl
from jax.experimental.pallas import tpu as pltpu
from jax.experimental.pallas import tpu_sc as plsc

N = 10000
T = 256
HID = 512
L = 256

BR = 1000

NW = 32
SHR = 10240
STRIPE = SHR // 16
BLK = 200

_mesh = plsc.VectorSubcoreMesh(core_axis_name="c", subcore_axis_name="s")


def _segsum_body(x_hbm, src_hbm, dst_hbm, outp_hbm,
                 idxbuf, dstbuf, rows, zbuf, shared, sem):
    E = src_hbm.shape[0]
    epw = E // NW
    c = lax.axis_index("c")
    s = lax.axis_index("s")
    w = s * 2 + c
    zbuf[...] = jnp.zeros_like(zbuf)
    for i in range(STRIPE // 64):
        pltpu.sync_copy(zbuf, shared.at[pl.ds(s * STRIPE + i * 64, 64)])
    plsc.subcore_barrier()
    base = w * epw

    def blk(b, _):
        off = pl.multiple_of(base + b * BLK, 8)
        pltpu.sync_copy(src_hbm.at[pl.ds(off, BLK)], idxbuf)
        pltpu.sync_copy(dst_hbm.at[pl.ds(off, BLK)], dstbuf)
        pltpu.async_copy(x_hbm.at[idxbuf], rows, sem).wait()
        pltpu.async_copy(rows, shared.at[dstbuf], sem, add=True).wait()
        return 0

    lax.fori_loop(0, epw // BLK, blk, 0)
    plsc.subcore_barrier()
    pltpu.sync_copy(shared.at[pl.ds(s * STRIPE, STRIPE)],
                    outp_hbm.at[pl.ds(pl.multiple_of(c * SHR + s * STRIPE, 8),
                                      STRIPE)])


def _segsum128(x, src, dst):
    k = pl.kernel(
        _segsum_body,
        mesh=_mesh,
        out_type=jax.ShapeDtypeStruct((2 * SHR, 128), jnp.float32),
        scratch_types=[pltpu.VMEM((BLK,), jnp.int32),
                       pltpu.VMEM((BLK,), jnp.int32),
                       pltpu.VMEM((BLK, 128), jnp.float32),
                       pltpu.VMEM((64, 128), jnp.float32),
                       pltpu.VMEM_SHARED((SHR, 128), jnp.float32),
                       pltpu.SemaphoreType.DMA],
    )
    outp = k(x, src, dst)
    return outp[:N], outp[SHR:SHR + N]


def _combine_body(a_ref, b_ref, o_ref):
    o_ref[...] = a_ref[...] + b_ref[...]


def _combine(a, b):
    n, w_ = a.shape
    return pl.pallas_call(
        _combine_body,
        grid=(n // BR,),
        in_specs=[pl.BlockSpec((BR, w_), lambda i: (i, 0)),
                  pl.BlockSpec((BR, w_), lambda i: (i, 0))],
        out_specs=pl.BlockSpec((BR, w_), lambda i: (i, 0)),
        out_shape=jax.ShapeDtypeStruct((n, w_), jnp.float32),
    )(a, b)


def _segsum(x, src, dst):
    cols = []
    for j in range(x.shape[1] // 128):
        p0, p1 = _segsum128(x[:, j * 128:(j + 1) * 128], src, dst)
        cols.append(_combine(p0, p1))
    return jnp.concatenate(cols, axis=1) if len(cols) > 1 else cols[0]


def _mm_stats_body(x_ref, agg_ref, w_ref, b_ref, h_ref, s1_ref, s2_ref):
    i = pl.program_id(0)
    u = x_ref[...] + agg_ref[...]
    h = jnp.dot(u, w_ref[...], preferred_element_type=jnp.float32) + b_ref[...]
    h_ref[...] = h

    @pl.when(i == 0)
    def _():
        s1_ref[...] = jnp.zeros_like(s1_ref)
        s2_ref[...] = jnp.zeros_like(s2_ref)

    s1_ref[...] += jnp.sum(h, axis=0, keepdims=True)
    s2_ref[...] += jnp.sum(h * h, axis=0, keepdims=True)


def _mm_stats(x, agg, w, b):
    n, k = x.shape
    c = w.shape[1]
    return pl.pallas_call(
        _mm_stats_body,
        grid=(n // BR,),
        in_specs=[
            pl.BlockSpec((BR, k), lambda i: (i, 0)),
            pl.BlockSpec((BR, k), lambda i: (i, 0)),
            pl.BlockSpec((k, c), lambda i: (0, 0)),
            pl.BlockSpec((1, c), lambda i: (0, 0)),
        ],
        out_specs=[
            pl.BlockSpec((BR, c), lambda i: (i, 0)),
            pl.BlockSpec((1, c), lambda i: (0, 0)),
            pl.BlockSpec((1, c), lambda i: (0, 0)),
        ],
        out_shape=[
            jax.ShapeDtypeStruct((n, c), jnp.float32),
            jax.ShapeDtypeStruct((1, c), jnp.float32),
            jax.ShapeDtypeStruct((1, c), jnp.float32),
        ],
    )(x, agg, w, b.reshape(1, c))


def _stats_body(s1_ref, s2_ref, g_ref, be_ref, a_ref, sh_ref):
    mean = s1_ref[...] * (1.0 / N)
    var = s2_ref[...] * (1.0 / N) - mean * mean
    a = g_ref[...] * jax.lax.rsqrt(var + 1e-5)
    a_ref[...] = a
    sh_ref[...] = be_ref[...] - mean * a


def _bn_scale(s1, s2, g, be):
    c = s1.shape[1]
    return pl.pallas_call(
        _stats_body,
        out_shape=[jax.ShapeDtypeStruct((1, c), jnp.float32),
                   jax.ShapeDtypeStruct((1, c), jnp.float32)],
    )(s1, s2, g.reshape(1, c), be.reshape(1, c))


def _bn_mm_body(h_ref, a_ref, sh_ref, w_ref, b_ref, o_ref, *, relu_out):
    t = jnp.maximum(h_ref[...] * a_ref[...] + sh_ref[...], 0.0)
    o = jnp.dot(t, w_ref[...], preferred_element_type=jnp.float32) + b_ref[...]
    if relu_out:
        o = jnp.maximum(o, 0.0)
    o_ref[...] = o


def _bn_mm(h, a, sh, w, b, relu_out):
    n, k = h.shape
    c = w.shape[1]
    return pl.pallas_call(
        functools.partial(_bn_mm_body, relu_out=relu_out),
        grid=(n // BR,),
        in_specs=[
            pl.BlockSpec((BR, k), lambda i: (i, 0)),
            pl.BlockSpec((1, k), lambda i: (0, 0)),
            pl.BlockSpec((1, k), lambda i: (0, 0)),
            pl.BlockSpec((k, c), lambda i: (0, 0)),
            pl.BlockSpec((1, c), lambda i: (0, 0)),
        ],
        out_specs=pl.BlockSpec((BR, c), lambda i: (i, 0)),
        out_shape=jax.ShapeDtypeStruct((n, c), jnp.float32),
    )(h, a, sh, w, b.reshape(1, c))


def _dual_mm_body(x_ref, wl_ref, bl_ref, wr_ref, br_ref, l_ref, r_ref):
    x = x_ref[...]
    l_ref[...] = jnp.dot(x, wl_ref[...], preferred_element_type=jnp.float32) + bl_ref[...]
    r_ref[...] = jnp.dot(x, wr_ref[...], preferred_element_type=jnp.float32) + br_ref[...]


def _dual_mm(x, wl, bl, wr, br):
    n, k = x.shape
    c = wl.shape[1]
    return pl.pallas_call(
        _dual_mm_body,
        grid=(n // BR,),
        in_specs=[
            pl.BlockSpec((BR, k), lambda i: (i, 0)),
            pl.BlockSpec((k, c), lambda i: (0, 0)),
            pl.BlockSpec((1, c), lambda i: (0, 0)),
            pl.BlockSpec((k, c), lambda i: (0, 0)),
            pl.BlockSpec((1, c), lambda i: (0, 0)),
        ],
        out_specs=[pl.BlockSpec((BR, c), lambda i: (i, 0)),
                   pl.BlockSpec((BR, c), lambda i: (i, 0))],
        out_shape=[jax.ShapeDtypeStruct((n, c), jnp.float32),
                   jax.ShapeDtypeStruct((n, c), jnp.float32)],
    )(x, wl, bl.reshape(1, c), wr, br.reshape(1, c))


def _final_body(flat_ref, wd_ref, bd_ref, out_ref):
    i = pl.program_id(0)

    @pl.when(i == 0)
    def _():
        out_ref[...] = jnp.zeros_like(out_ref)

    out_ref[...] += jnp.sum(flat_ref[...] * wd_ref[...]).reshape(1, 1)

    @pl.when(i == pl.num_programs(0) - 1)
    def _():
        out_ref[...] = jax.nn.sigmoid(out_ref[...] + bd_ref[...])


def _final_dot(flat, wd, bd):
    return pl.pallas_call(
        _final_body,
        grid=(N // BR,),
        in_specs=[pl.BlockSpec((BR, L), lambda i: (i, 0)),
                  pl.BlockSpec((BR, L), lambda i: (i, 0)),
                  pl.BlockSpec((1, 1), lambda i: (0, 0))],
        out_specs=pl.BlockSpec((1, 1), lambda i: (0, 0)),
        out_shape=jax.ShapeDtypeStruct((1, 1), jnp.float32),
    )(flat, wd, bd.reshape(1, 1))


def _gin_conv(x, agg, W1, b1, g, be, W2, b2, relu_out):
    h, s1, s2 = _mm_stats(x, agg, W1, b1)
    a, sh = _bn_scale(s1, s2, g, be)
    return _bn_mm(h, a, sh, W2, b2, relu_out)


def _gatv2_edges(xl, xr, s, d, att, heads, out_ch):
    e = jax.nn.leaky_relu(xl[s] + xr[d], 0.2)
    alpha = (e * att).sum(-1)
    m = jax.ops.segment_max(alpha, d, num_segments=N)
    alpha = jnp.exp(alpha - m[d])
    denom = jax.ops.segment_sum(alpha, d, num_segments=N)
    alpha = alpha / (denom[d] + 1e-16)
    return jax.ops.segment_sum(xl[s] * alpha[:, :, None], d, num_segments=N)


def kernel(eeg_nodes, eeg_idx, W11, b11, g1, be1, W12, b12, W21, b21, g2, be2, W22, b22,
           Wl1, bl1, Wr1, br1, att1, bias1, Wl2, bl2, Wr2, br2, att2, bias2, Wd, bd):
    src = eeg_idx[0].astype(jnp.int32)
    dst = eeg_idx[1].astype(jnp.int32)
    loop = jnp.arange(N, dtype=src.dtype)
    s2_ = jnp.concatenate([src, loop])
    d2_ = jnp.concatenate([dst, loop])

    agg1 = _segsum(eeg_nodes, src, dst)
    h = _gin_conv(eeg_nodes, agg1, W11, b11, g1, be1, W12, b12, relu_out=True)
    agg2 = _segsum(h, src, dst)
    h = _gin_conv(h, agg2, W21, b21, g2, be2, W22, b22, relu_out=False)

    xl1, xr1 = _dual_mm(h, Wl1, bl1, Wr1, br1)
    xl1 = xl1.reshape(N, 4, L)
    xr1 = xr1.reshape(N, 4, L)
    r1 = _gatv2_edges(xl1, xr1, s2_, d2_, att1, 4, L).reshape(N, 4 * L) + bias1

    xl2 = (r1 @ Wl2 + bl2).reshape(N, 1, 1)
    xr2 = (r1 @ Wr2 + br2).reshape(N, 1, 1)
    region_scores = _gatv2_edges(xl2, xr2, s2_, d2_, att2, 1, 1).reshape(N, 1) + bias2

    dementia_pred = _final_dot(h, Wd.reshape(N, L), bd)
    return (dementia_pred, region_scores)


# speedup vs baseline: 2.9495x; 2.7503x over previous
import functools
import jax
import jax.numpy as jnp
from jax import lax
from jax.experimental import pallas as pl
from jax.experimental.pallas import tpu as pltpu
from jax.experimental.pallas import tpu_sc as plsc

N = 10000
T = 256
HID = 512
L = 256

BR = 1000

NW = 32
SHR = 10240
STRIPE = SHR // 16
BLK = 200

_mesh = plsc.VectorSubcoreMesh(core_axis_name="c", subcore_axis_name="s")


def _iota16():
    return lax.iota(jnp.int32, 16)


def _segsum_body(x_hbm, src_hbm, dst_hbm, outp_hbm,
                 idxbuf, dstbuf, rows, zbuf, shared, sem):
    E = src_hbm.shape[0]
    epw = E // NW
    c = lax.axis_index("c")
    s = lax.axis_index("s")
    w = s * 2 + c
    zbuf[...] = jnp.zeros_like(zbuf)
    for i in range(STRIPE // 64):
        pltpu.sync_copy(zbuf, shared.at[pl.ds(s * STRIPE + i * 64, 64)])
    plsc.subcore_barrier()
    base = w * epw

    def blk(b, _):
        off = pl.multiple_of(base + b * BLK, 8)
        pltpu.sync_copy(src_hbm.at[pl.ds(off, BLK)], idxbuf)
        pltpu.sync_copy(dst_hbm.at[pl.ds(off, BLK)], dstbuf)
        pltpu.async_copy(x_hbm.at[idxbuf], rows, sem).wait()
        pltpu.async_copy(rows, shared.at[dstbuf], sem, add=True).wait()
        return 0

    lax.fori_loop(0, epw // BLK, blk, 0)
    plsc.subcore_barrier()
    pltpu.sync_copy(shared.at[pl.ds(s * STRIPE, STRIPE)],
                    outp_hbm.at[pl.ds(pl.multiple_of(c * SHR + s * STRIPE, 8),
                                      STRIPE)])


def _segsum128(x, src, dst):
    k = pl.kernel(
        _segsum_body,
        mesh=_mesh,
        out_type=jax.ShapeDtypeStruct((2 * SHR, 128), jnp.float32),
        scratch_types=[pltpu.VMEM((BLK,), jnp.int32),
                       pltpu.VMEM((BLK,), jnp.int32),
                       pltpu.VMEM((BLK, 128), jnp.float32),
                       pltpu.VMEM((64, 128), jnp.float32),
                       pltpu.VMEM_SHARED((SHR, 128), jnp.float32),
                       pltpu.SemaphoreType.DMA],
    )
    outp = k(x, src, dst)
    return outp[:N], outp[SHR:SHR + N]


def _combine_body(a_ref, b_ref, o_ref):
    o_ref[...] = a_ref[...] + b_ref[...]


def _combine(a, b):
    n, w_ = a.shape
    return pl.pallas_call(
        _combine_body,
        grid=(n // BR,),
        in_specs=[pl.BlockSpec((BR, w_), lambda i: (i, 0)),
                  pl.BlockSpec((BR, w_), lambda i: (i, 0))],
        out_specs=pl.BlockSpec((BR, w_), lambda i: (i, 0)),
        out_shape=jax.ShapeDtypeStruct((n, w_), jnp.float32),
    )(a, b)


def _segsum(x, src, dst):
    cols = []
    for j in range(x.shape[1] // 128):
        p0, p1 = _segsum128(x[:, j * 128:(j + 1) * 128], src, dst)
        cols.append(_combine(p0, p1))
    return jnp.concatenate(cols, axis=1) if len(cols) > 1 else cols[0]


GBLK = 40
GBLK1 = 16


def _zero_shared(s, zbuf, shared):
    zbuf[...] = jnp.zeros_like(zbuf)
    for i in range(STRIPE // 64):
        pltpu.sync_copy(zbuf, shared.at[pl.ds(s * STRIPE + i * 64, 64)])
    plsc.subcore_barrier()


def _writeback(c, s, shared, outp_hbm):
    plsc.subcore_barrier()
    pltpu.sync_copy(shared.at[pl.ds(s * STRIPE, STRIPE)],
                    outp_hbm.at[pl.ds(pl.multiple_of(c * SHR + s * STRIPE, 8),
                                      STRIPE)])


def _zero_lrow_tail(lrow, nblk):
    def z(e, _):
        for cc in range(1, 8):
            lrow[e, pl.ds(cc * 16, 16)] = jnp.zeros((16,), jnp.float32)
        return 0

    lax.fori_loop(0, nblk, z, 0)


def _gat1_logits(rs, rd, attv, padf, e):
    louts = []
    for h in range(4):
        acc = jnp.zeros((16,), jnp.float32)
        for cc in range(16):
            o = h * 256 + cc * 16
            z = rs[e, pl.ds(o, 16)] + rd[e, pl.ds(o, 16)]
            lr = jnp.maximum(z, 0.2 * z)
            acc = acc + lr * attv[pl.ds(o, 16)]
        pre = acc
        for sh in (1, 2, 4, 8):
            padf[pl.ds(16, 16)] = pre
            pre = pre + padf[pl.ds(16 - sh, 16)]
        louts.append(pre[15])
    return louts


def _gat1_mean_body(xl_hbm, xrp_hbm, src_hbm, dst_hbm, att_hbm, outp_hbm,
                    sbuf, dbuf, rs, rd, attv, lrow, padf, zbuf, shared, sem):
    E2P = src_hbm.shape[0]
    epw = E2P // NW
    c = lax.axis_index("c")
    s = lax.axis_index("s")
    w = s * 2 + c
    iota = _iota16()
    _zero_shared(s, zbuf, shared)
    _zero_lrow_tail(lrow, GBLK1)
    padf[pl.ds(0, 16)] = jnp.zeros((16,), jnp.float32)
    pltpu.sync_copy(att_hbm, attv)
    base = w * epw

    def blk(b, _):
        off = pl.multiple_of(base + b * GBLK1, 8)
        pltpu.sync_copy(src_hbm.at[pl.ds(off, GBLK1)], sbuf)
        pltpu.sync_copy(dst_hbm.at[pl.ds(off, GBLK1)], dbuf)
        pltpu.async_copy(xl_hbm.at[sbuf], rs, sem).wait()
        pltpu.async_copy(xrp_hbm.at[dbuf], rd, sem).wait()

        def edge(e, _):
            louts = _gat1_logits(rs, rd, attv, padf, e)
            row = jnp.where(iota == 4, 1.0, 0.0).astype(jnp.float32)
            for h in range(4):
                row = jnp.where(iota == h, louts[h], row)
            lrow[e, pl.ds(0, 16)] = row
            return 0

        lax.fori_loop(0, GBLK1, edge, 0)
        pltpu.async_copy(lrow, shared.at[dbuf], sem, add=True).wait()
        return 0

    lax.fori_loop(0, epw // GBLK1, blk, 0)
    _writeback(c, s, shared, outp_hbm)


def _gat1_wts_body(xl_hbm, xrp_hbm, src_hbm, dst_hbm, att_hbm, meanp_hbm,
                   outp_hbm, w_hbm,
                   sbuf, dbuf, rs, rd, mr, attv, lrow, wstage, padf, zbuf,
                   shared, sem):
    E2P = src_hbm.shape[0]
    epw = E2P // NW
    c = lax.axis_index("c")
    s = lax.axis_index("s")
    w = s * 2 + c
    iota = _iota16()
    _zero_shared(s, zbuf, shared)
    _zero_lrow_tail(lrow, GBLK1)
    padf[pl.ds(0, 16)] = jnp.zeros((16,), jnp.float32)
    pltpu.sync_copy(att_hbm, attv)
    base = w * epw
    zf = jnp.zeros((16,), jnp.float32)

    def blk(b, _):
        off = pl.multiple_of(base + b * GBLK1, 8)
        pltpu.sync_copy(src_hbm.at[pl.ds(off, GBLK1)], sbuf)
        pltpu.sync_copy(dst_hbm.at[pl.ds(off, GBLK1)], dbuf)
        pltpu.async_copy(xl_hbm.at[sbuf], rs, sem).wait()
        pltpu.async_copy(xrp_hbm.at[dbuf], rd, sem).wait()
        pltpu.async_copy(meanp_hbm.at[dbuf], mr, sem).wait()

        def edge(e, _):
            louts = _gat1_logits(rs, rd, attv, padf, e)
            mrow = mr[e, pl.ds(0, 16)]
            row = zf
            for h in range(4):
                wv = jnp.exp(jnp.minimum(zf + (louts[h] - mrow[h]), 75.0))
                row = jnp.where(iota == h, wv, row)
            lrow[e, pl.ds(0, 16)] = row
            wstage[pl.ds(e * 16, 16)] = row
            return 0

        lax.fori_loop(0, GBLK1, edge, 0)
        pltpu.async_copy(lrow, shared.at[dbuf], sem, add=True).wait()
        pltpu.sync_copy(wstage,
                        w_hbm.at[pl.ds(pl.multiple_of(off * 16, 8), GBLK1 * 16)])
        return 0

    lax.fori_loop(0, epw // GBLK1, blk, 0)
    _writeback(c, s, shared, outp_hbm)


def _wseg_body(xcol_hbm, src_hbm, dst_hbm, w_hbm, outp_hbm,
               sbuf, dbuf, wbuf, rows, zbuf, shared, sem, *, hlane):
    E2P = src_hbm.shape[0]
    epw = E2P // NW
    c = lax.axis_index("c")
    s = lax.axis_index("s")
    w = s * 2 + c
    _zero_shared(s, zbuf, shared)
    base = w * epw

    def blk(b, _):
        off = pl.multiple_of(base + b * GBLK, 8)
        pltpu.sync_copy(src_hbm.at[pl.ds(off, GBLK)], sbuf)
        pltpu.sync_copy(dst_hbm.at[pl.ds(off, GBLK)], dbuf)
        pltpu.sync_copy(w_hbm.at[pl.ds(pl.multiple_of(off * 16, 8), GBLK * 16)],
                        wbuf)
        pltpu.async_copy(xcol_hbm.at[sbuf], rows, sem).wait()

        def edge(e, _):
            wv = wbuf[pl.ds(e * 16, 16)]
            ws = wv[hlane]
            for cc in range(8):
                rows[e, pl.ds(cc * 16, 16)] = rows[e, pl.ds(cc * 16, 16)] * ws
            return 0

        lax.fori_loop(0, GBLK, edge, 0)
        pltpu.async_copy(rows, shared.at[dbuf], sem, add=True).wait()
        return 0

    lax.fori_loop(0, epw // GBLK, blk, 0)
    _writeback(c, s, shared, outp_hbm)


def _gat2_body(xla_hbm, xrp_hbm, src_hbm, dst_hbm, att_hbm, meanp_hbm,
               outp_hbm, sbuf, dbuf, xa, xb, mr, attv, lrow, zbuf, shared,
               sem, *, mode):
    E2P = src_hbm.shape[0]
    epw = E2P // NW
    c = lax.axis_index("c")
    s = lax.axis_index("s")
    w = s * 2 + c
    iota = _iota16()
    _zero_shared(s, zbuf, shared)
    _zero_lrow_tail(lrow, GBLK)
    pltpu.sync_copy(att_hbm, attv)
    att2s = attv[...][0]
    base = w * epw
    zf = jnp.zeros((16,), jnp.float32)

    def blk(b, _):
        off = pl.multiple_of(base + b * GBLK, 8)
        pltpu.sync_copy(src_hbm.at[pl.ds(off, GBLK)], sbuf)
        pltpu.sync_copy(dst_hbm.at[pl.ds(off, GBLK)], dbuf)
        pltpu.async_copy(xla_hbm.at[sbuf], xa, sem).wait()
        pltpu.async_copy(xrp_hbm.at[dbuf], xb, sem).wait()
        if mode == "num":
            pltpu.async_copy(meanp_hbm.at[dbuf], mr, sem).wait()

        def edge(e, _):
            a0 = xa[e, pl.ds(0, 16)]
            b0 = xb[e, pl.ds(0, 16)]
            z = a0 + b0
            lr = jnp.maximum(z, 0.2 * z)
            lv = lr * att2s
            if mode == "mean":
                row = lv + jnp.where(iota == 1, 1.0, 0.0).astype(jnp.float32)
            else:
                m0 = mr[e, pl.ds(0, 16)]
                wv = jnp.exp(jnp.minimum(zf + (lv[0] - m0[0]), 75.0))
                row = jnp.where(iota == 0, wv * a0[0],
                                jnp.where(iota == 1, wv, zf)).astype(
                                    jnp.float32)
            lrow[e, pl.ds(0, 16)] = row
            return 0

        lax.fori_loop(0, GBLK, edge, 0)
        pltpu.async_copy(lrow, shared.at[dbuf], sem, add=True).wait()
        return 0

    lax.fori_loop(0, epw // GBLK, blk, 0)
    _writeback(c, s, shared, outp_hbm)


def _gat1_mean(xl, xrp, src, dst, attf):
    k = pl.kernel(
        _gat1_mean_body,
        mesh=_mesh,
        out_type=jax.ShapeDtypeStruct((2 * SHR, 128), jnp.float32),
        scratch_types=[pltpu.VMEM((GBLK1,), jnp.int32),
                       pltpu.VMEM((GBLK1,), jnp.int32),
                       pltpu.VMEM((GBLK1, 1024), jnp.float32),
                       pltpu.VMEM((GBLK1, 1024), jnp.float32),
                       pltpu.VMEM((1024,), jnp.float32),
                       pltpu.VMEM((GBLK1, 128), jnp.float32),
                       pltpu.VMEM((32,), jnp.float32),
                       pltpu.VMEM((64, 128), jnp.float32),
                       pltpu.VMEM_SHARED((SHR, 128), jnp.float32),
                       pltpu.SemaphoreType.DMA],
    )
    return k(xl, xrp, src, dst, attf)


def _gat1_wts(xl, xrp, src, dst, attf, meanp, e2p):
    k = pl.kernel(
        _gat1_wts_body,
        mesh=_mesh,
        out_type=[jax.ShapeDtypeStruct((2 * SHR, 128), jnp.float32),
                  jax.ShapeDtypeStruct((e2p * 16,), jnp.float32)],
        scratch_types=[pltpu.VMEM((GBLK1,), jnp.int32),
                       pltpu.VMEM((GBLK1,), jnp.int32),
                       pltpu.VMEM((GBLK1, 1024), jnp.float32),
                       pltpu.VMEM((GBLK1, 1024), jnp.float32),
                       pltpu.VMEM((GBLK1, 128), jnp.float32),
                       pltpu.VMEM((1024,), jnp.float32),
                       pltpu.VMEM((GBLK1, 128), jnp.float32),
                       pltpu.VMEM((GBLK1 * 16,), jnp.float32),
                       pltpu.VMEM((32,), jnp.float32),
                       pltpu.VMEM((64, 128), jnp.float32),
                       pltpu.VMEM_SHARED((SHR, 128), jnp.float32),
                       pltpu.SemaphoreType.DMA],
    )
    return k(xl, xrp, src, dst, attf, meanp)


def _wseg(xcol, src, dst, wts, hlane):
    k = pl.kernel(
        functools.partial(_wseg_body, hlane=hlane),
        mesh=_mesh,
        out_type=jax.ShapeDtypeStruct((2 * SHR, 128), jnp.float32),
        scratch_types=[pltpu.VMEM((GBLK,), jnp.int32),
                       pltpu.VMEM((GBLK,), jnp.int32),
                       pltpu.VMEM((GBLK * 16,), jnp.float32),
                       pltpu.VMEM((GBLK, 128), jnp.float32),
                       pltpu.VMEM((64, 128), jnp.float32),
                       pltpu.VMEM_SHARED((SHR, 128), jnp.float32),
                       pltpu.SemaphoreType.DMA],
    )
    return k(xcol, src, dst, wts)


def _gat2(xla, xrp, src, dst, att2f, meanp, mode):
    k = pl.kernel(
        functools.partial(_gat2_body, mode=mode),
        mesh=_mesh,
        out_type=jax.ShapeDtypeStruct((2 * SHR, 128), jnp.float32),
        scratch_types=[pltpu.VMEM((GBLK,), jnp.int32),
                       pltpu.VMEM((GBLK,), jnp.int32),
                       pltpu.VMEM((GBLK, 128), jnp.float32),
                       pltpu.VMEM((GBLK, 128), jnp.float32),
                       pltpu.VMEM((GBLK, 128), jnp.float32),
                       pltpu.VMEM((16,), jnp.float32),
                       pltpu.VMEM((GBLK, 128), jnp.float32),
                       pltpu.VMEM((64, 128), jnp.float32),
                       pltpu.VMEM_SHARED((SHR, 128), jnp.float32),
                       pltpu.SemaphoreType.DMA],
    )
    return k(xla, xrp, src, dst, att2f, meanp)


def _mm_stats_body(x_ref, agg_ref, w_ref, b_ref, h_ref, s1_ref, s2_ref):
    i = pl.program_id(0)
    u = x_ref[...] + agg_ref[...]
    h = jnp.dot(u, w_ref[...], preferred_element_type=jnp.float32) + b_ref[...]
    h_ref[...] = h

    @pl.when(i == 0)
    def _():
        s1_ref[...] = jnp.zeros_like(s1_ref)
        s2_ref[...] = jnp.zeros_like(s2_ref)

    s1_ref[...] += jnp.sum(h, axis=0, keepdims=True)
    s2_ref[...] += jnp.sum(h * h, axis=0, keepdims=True)


def _mm_stats(x, agg, w, b):
    n, k = x.shape
    c = w.shape[1]
    return pl.pallas_call(
        _mm_stats_body,
        grid=(n // BR,),
        in_specs=[
            pl.BlockSpec((BR, k), lambda i: (i, 0)),
            pl.BlockSpec((BR, k), lambda i: (i, 0)),
            pl.BlockSpec((k, c), lambda i: (0, 0)),
            pl.BlockSpec((1, c), lambda i: (0, 0)),
        ],
        out_specs=[
            pl.BlockSpec((BR, c), lambda i: (i, 0)),
            pl.BlockSpec((1, c), lambda i: (0, 0)),
            pl.BlockSpec((1, c), lambda i: (0, 0)),
        ],
        out_shape=[
            jax.ShapeDtypeStruct((n, c), jnp.float32),
            jax.ShapeDtypeStruct((1, c), jnp.float32),
            jax.ShapeDtypeStruct((1, c), jnp.float32),
        ],
    )(x, agg, w, b.reshape(1, c))


def _stats_body(s1_ref, s2_ref, g_ref, be_ref, a_ref, sh_ref):
    mean = s1_ref[...] * (1.0 / N)
    var = s2_ref[...] * (1.0 / N) - mean * mean
    a = g_ref[...] * jax.lax.rsqrt(var + 1e-5)
    a_ref[...] = a
    sh_ref[...] = be_ref[...] - mean * a


def _bn_scale(s1, s2, g, be):
    c = s1.shape[1]
    return pl.pallas_call(
        _stats_body,
        out_shape=[jax.ShapeDtypeStruct((1, c), jnp.float32),
                   jax.ShapeDtypeStruct((1, c), jnp.float32)],
    )(s1, s2, g.reshape(1, c), be.reshape(1, c))


def _bn_mm_body(h_ref, a_ref, sh_ref, w_ref, b_ref, o_ref, *, relu_out):
    t = jnp.maximum(h_ref[...] * a_ref[...] + sh_ref[...], 0.0)
    o = jnp.dot(t, w_ref[...], preferred_element_type=jnp.float32) + b_ref[...]
    if relu_out:
        o = jnp.maximum(o, 0.0)
    o_ref[...] = o


def _bn_mm(h, a, sh, w, b, relu_out):
    n, k = h.shape
    c = w.shape[1]
    return pl.pallas_call(
        functools.partial(_bn_mm_body, relu_out=relu_out),
        grid=(n // BR,),
        in_specs=[
            pl.BlockSpec((BR, k), lambda i: (i, 0)),
            pl.BlockSpec((1, k), lambda i: (0, 0)),
            pl.BlockSpec((1, k), lambda i: (0, 0)),
            pl.BlockSpec((k, c), lambda i: (0, 0)),
            pl.BlockSpec((1, c), lambda i: (0, 0)),
        ],
        out_specs=pl.BlockSpec((BR, c), lambda i: (i, 0)),
        out_shape=jax.ShapeDtypeStruct((n, c), jnp.float32),
    )(h, a, sh, w, b.reshape(1, c))


def _dual_mm_body(x_ref, wl_ref, bl_ref, wr_ref, br_ref, l_ref, r_ref):
    x = x_ref[...]
    l_ref[...] = jnp.dot(x, wl_ref[...], preferred_element_type=jnp.float32) + bl_ref[...]
    r_ref[...] = jnp.dot(x, wr_ref[...], preferred_element_type=jnp.float32) + br_ref[...]


def _dual_mm(x, wl, bl, wr, br):
    n, k = x.shape
    c = wl.shape[1]
    return pl.pallas_call(
        _dual_mm_body,
        grid=(n // BR,),
        in_specs=[
            pl.BlockSpec((BR, k), lambda i: (i, 0)),
            pl.BlockSpec((k, c), lambda i: (0, 0)),
            pl.BlockSpec((1, c), lambda i: (0, 0)),
            pl.BlockSpec((k, c), lambda i: (0, 0)),
            pl.BlockSpec((1, c), lambda i: (0, 0)),
        ],
        out_specs=[pl.BlockSpec((BR, c), lambda i: (i, 0)),
                   pl.BlockSpec((BR, c), lambda i: (i, 0))],
        out_shape=[jax.ShapeDtypeStruct((n, c), jnp.float32),
                   jax.ShapeDtypeStruct((n, c), jnp.float32)],
    )(x, wl, bl.reshape(1, c), wr, br.reshape(1, c))


def _final_body(flat_ref, wd_ref, bd_ref, out_ref):
    i = pl.program_id(0)

    @pl.when(i == 0)
    def _():
        out_ref[...] = jnp.zeros_like(out_ref)

    out_ref[...] += jnp.sum(flat_ref[...] * wd_ref[...]).reshape(1, 1)

    @pl.when(i == pl.num_programs(0) - 1)
    def _():
        out_ref[...] = jax.nn.sigmoid(out_ref[...] + bd_ref[...])


def _final_dot(flat, wd, bd):
    return pl.pallas_call(
        _final_body,
        grid=(N // BR,),
        in_specs=[pl.BlockSpec((BR, L), lambda i: (i, 0)),
                  pl.BlockSpec((BR, L), lambda i: (i, 0)),
                  pl.BlockSpec((1, 1), lambda i: (0, 0))],
        out_specs=pl.BlockSpec((1, 1), lambda i: (0, 0)),
        out_shape=jax.ShapeDtypeStruct((1, 1), jnp.float32),
    )(flat, wd, bd.reshape(1, 1))


def _mean1_body(p0_ref, p1_ref, o_ref):
    ps = p0_ref[...] + p1_ref[...]
    cnt = jnp.maximum(ps[:, 4:5], 1.0)
    m = ps[:, 0:4] / cnt
    o_ref[...] = jnp.concatenate(
        [m, jnp.zeros((m.shape[0], 124), jnp.float32)], axis=1)


def _mean1(p0, p1):
    return pl.pallas_call(
        _mean1_body,
        grid=(N // BR,),
        in_specs=[pl.BlockSpec((BR, 128), lambda i: (i, 0)),
                  pl.BlockSpec((BR, 128), lambda i: (i, 0))],
        out_specs=pl.BlockSpec((BR, 128), lambda i: (i, 0)),
        out_shape=jax.ShapeDtypeStruct((N, 128), jnp.float32),
    )(p0, p1)


def _gat1_norm_body(n_ref, d0_ref, d1_ref, b_ref, o_ref):
    den = (d0_ref[...] + d1_ref[...])[:, 0:4] + 1e-16
    scale = jnp.repeat(1.0 / den, 256, axis=1)
    o_ref[...] = n_ref[...] * scale + b_ref[...]


def _gat1_norm(num, d0, d1, bias1):
    return pl.pallas_call(
        _gat1_norm_body,
        grid=(N // BR,),
        in_specs=[pl.BlockSpec((BR, 1024), lambda i: (i, 0)),
                  pl.BlockSpec((BR, 128), lambda i: (i, 0)),
                  pl.BlockSpec((BR, 128), lambda i: (i, 0)),
                  pl.BlockSpec((1, 1024), lambda i: (0, 0))],
        out_specs=pl.BlockSpec((BR, 1024), lambda i: (i, 0)),
        out_shape=jax.ShapeDtypeStruct((N, 1024), jnp.float32),
    )(num, d0, d1, bias1.reshape(1, 1024))


def _gat2_proj_body(x_ref, wl_ref, bl_ref, wr_ref, br_ref, la_ref, ra_ref):
    x = x_ref[...]
    z = jnp.zeros((x.shape[0], 127), jnp.float32)
    xl2 = jnp.dot(x, wl_ref[...], preferred_element_type=jnp.float32) + bl_ref[...]
    xr2 = jnp.dot(x, wr_ref[...], preferred_element_type=jnp.float32) + br_ref[...]
    la_ref[...] = jnp.concatenate([xl2, z], axis=1)
    ra_ref[...] = jnp.concatenate([xr2, z], axis=1)


def _gat2_proj(r1, wl2, bl2, wr2, br2):
    return pl.pallas_call(
        _gat2_proj_body,
        grid=(N // BR,),
        in_specs=[pl.BlockSpec((BR, 1024), lambda i: (i, 0)),
                  pl.BlockSpec((1024, 1), lambda i: (0, 0)),
                  pl.BlockSpec((1, 1), lambda i: (0, 0)),
                  pl.BlockSpec((1024, 1), lambda i: (0, 0)),
                  pl.BlockSpec((1, 1), lambda i: (0, 0))],
        out_specs=[pl.BlockSpec((BR, 128), lambda i: (i, 0)),
                   pl.BlockSpec((BR, 128), lambda i: (i, 0))],
        out_shape=[jax.ShapeDtypeStruct((N, 128), jnp.float32),
                   jax.ShapeDtypeStruct((N, 128), jnp.float32)],
    )(r1, wl2, bl2.reshape(1, 1), wr2, br2.reshape(1, 1))


def _gat2_mean_body(p0_ref, p1_ref, o_ref):
    ps = p0_ref[...] + p1_ref[...]
    m = ps[:, 0:1] / jnp.maximum(ps[:, 1:2], 1.0)
    o_ref[...] = jnp.concatenate(
        [m, jnp.zeros((m.shape[0], 127), jnp.float32)], axis=1)


def _gat2_mean(p0, p1):
    return pl.pallas_call(
        _gat2_mean_body,
        grid=(N // BR,),
        in_specs=[pl.BlockSpec((BR, 128), lambda i: (i, 0)),
                  pl.BlockSpec((BR, 128), lambda i: (i, 0))],
        out_specs=pl.BlockSpec((BR, 128), lambda i: (i, 0)),
        out_shape=jax.ShapeDtypeStruct((N, 128), jnp.float32),
    )(p0, p1)


def _gat2_score_body(p0_ref, p1_ref, b_ref, o_ref):
    ps = p0_ref[...] + p1_ref[...]
    sc = ps[:, 0:1] / (ps[:, 1:2] + 1e-16) + b_ref[...]
    o_ref[...] = jnp.concatenate(
        [sc, jnp.zeros((sc.shape[0], 127), jnp.float32)], axis=1)


def _gat2_score(p0, p1, bias2):
    return pl.pallas_call(
        _gat2_score_body,
        grid=(N // BR,),
        in_specs=[pl.BlockSpec((BR, 128), lambda i: (i, 0)),
                  pl.BlockSpec((BR, 128), lambda i: (i, 0)),
                  pl.BlockSpec((1, 1), lambda i: (0, 0))],
        out_specs=pl.BlockSpec((BR, 128), lambda i: (i, 0)),
        out_shape=jax.ShapeDtypeStruct((N, 128), jnp.float32),
    )(p0, p1, bias2.reshape(1, 1))


def _gin_conv(x, agg, W1, b1, g, be, W2, b2, relu_out):
    h, s1, s2 = _mm_stats(x, agg, W1, b1)
    a, sh = _bn_scale(s1, s2, g, be)
    return _bn_mm(h, a, sh, W2, b2, relu_out)


def kernel(eeg_nodes, eeg_idx, W11, b11, g1, be1, W12, b12, W21, b21, g2, be2, W22, b22,
           Wl1, bl1, Wr1, br1, att1, bias1, Wl2, bl2, Wr2, br2, att2, bias2, Wd, bd):
    src = eeg_idx[0].astype(jnp.int32)
    dst = eeg_idx[1].astype(jnp.int32)
    E = src.shape[0]
    loop = jnp.arange(N, dtype=jnp.int32)
    E2 = E + N
    E2P = -(-E2 // (NW * 80)) * (NW * 80)
    s2p = jnp.concatenate([src, loop, jnp.zeros((E2P - E2,), jnp.int32)])
    d2p = jnp.concatenate([dst, loop,
                           jnp.full((E2P - E2,), SHR - 1, jnp.int32)])

    agg1 = _segsum(eeg_nodes, src, dst)
    h = _gin_conv(eeg_nodes, agg1, W11, b11, g1, be1, W12, b12, relu_out=True)
    agg2 = _segsum(h, src, dst)
    h = _gin_conv(h, agg2, W21, b21, g2, be2, W22, b22, relu_out=False)

    xl1, xr1 = _dual_mm(h, Wl1, bl1, Wr1, br1)
    xr1p = jnp.pad(xr1, ((0, SHR - N), (0, 0)))
    attf = att1.reshape(1024)
    mp = _gat1_mean(xl1, xr1p, s2p, d2p, attf)
    meanp = jnp.pad(_mean1(mp[:N], mp[SHR:SHR + N]), ((0, SHR - N), (0, 0)))
    wp, wts = _gat1_wts(xl1, xr1p, s2p, d2p, attf, meanp, E2P)
    cols = []
    for j in range(8):
        pj = _wseg(xl1[:, j * 128:(j + 1) * 128], s2p, d2p, wts, j // 2)
        cols.append(_combine(pj[:N], pj[SHR:SHR + N]))
    num = jnp.concatenate(cols, axis=1)
    r1 = _gat1_norm(num, wp[:N], wp[SHR:SHR + N], bias1)

    xla, xra = _gat2_proj(r1, Wl2, bl2, Wr2, br2)
    xrap = jnp.pad(xra, ((0, SHR - N), (0, 0)))
    att2f = jnp.pad(att2.reshape(1), (0, 15))
    m2p = _gat2(xla, xrap, s2p, d2p, att2f, xrap, mode="mean")
    mean2p = jnp.pad(_gat2_mean(m2p[:N], m2p[SHR:SHR + N]),
                     ((0, SHR - N), (0, 0)))
    q = _gat2(xla, xrap, s2p, d2p, att2f, mean2p, mode="num")
    region_scores = _gat2_score(q[:N], q[SHR:SHR + N], bias2)[:, :1]

    dementia_pred = _final_dot(h, Wd.reshape(N, L), bd)
    return (dementia_pred, region_scores)


# speedup vs baseline: 3.5727x; 1.2113x over previous
import functools
import jax
import jax.numpy as jnp
from jax import lax
from jax.experimental import pallas as pl
from jax.experimental.pallas import tpu as pltpu
from jax.experimental.pallas import tpu_sc as plsc

N = 10000
T = 256
HID = 512
L = 256

BR = 1000

NW = 32
SHR = 10240
STRIPE = SHR // 16
BLK = 200

_mesh = plsc.VectorSubcoreMesh(core_axis_name="c", subcore_axis_name="s")


def _iota16():
    return lax.iota(jnp.int32, 16)


def _segsum_body(x_hbm, src_hbm, dst_hbm, outp_hbm,
                 idxbuf, dstbuf, rows, zbuf, shared, sem):
    E = src_hbm.shape[0]
    epw = E // NW
    c = lax.axis_index("c")
    s = lax.axis_index("s")
    w = s * 2 + c
    zbuf[...] = jnp.zeros_like(zbuf)
    for i in range(STRIPE // 64):
        pltpu.sync_copy(zbuf, shared.at[pl.ds(s * STRIPE + i * 64, 64)])
    plsc.subcore_barrier()
    base = w * epw

    def blk(b, _):
        off = pl.multiple_of(base + b * BLK, 8)
        pltpu.sync_copy(src_hbm.at[pl.ds(off, BLK)], idxbuf)
        pltpu.sync_copy(dst_hbm.at[pl.ds(off, BLK)], dstbuf)
        pltpu.async_copy(x_hbm.at[idxbuf], rows, sem).wait()
        pltpu.async_copy(rows, shared.at[dstbuf], sem, add=True).wait()
        return 0

    lax.fori_loop(0, epw // BLK, blk, 0)
    plsc.subcore_barrier()
    pltpu.sync_copy(shared.at[pl.ds(s * STRIPE, STRIPE)],
                    outp_hbm.at[pl.ds(pl.multiple_of(c * SHR + s * STRIPE, 8),
                                      STRIPE)])


def _segsum128(x, src, dst):
    k = pl.kernel(
        _segsum_body,
        mesh=_mesh,
        out_type=jax.ShapeDtypeStruct((2 * SHR, 128), jnp.float32),
        scratch_types=[pltpu.VMEM((BLK,), jnp.int32),
                       pltpu.VMEM((BLK,), jnp.int32),
                       pltpu.VMEM((BLK, 128), jnp.float32),
                       pltpu.VMEM((64, 128), jnp.float32),
                       pltpu.VMEM_SHARED((SHR, 128), jnp.float32),
                       pltpu.SemaphoreType.DMA],
    )
    outp = k(x, src, dst)
    return outp[:N], outp[SHR:SHR + N]


def _combine_body(a_ref, b_ref, o_ref):
    o_ref[...] = a_ref[...] + b_ref[...]


def _combine(a, b):
    n, w_ = a.shape
    return pl.pallas_call(
        _combine_body,
        grid=(n // BR,),
        in_specs=[pl.BlockSpec((BR, w_), lambda i: (i, 0)),
                  pl.BlockSpec((BR, w_), lambda i: (i, 0))],
        out_specs=pl.BlockSpec((BR, w_), lambda i: (i, 0)),
        out_shape=jax.ShapeDtypeStruct((n, w_), jnp.float32),
    )(a, b)


def _segsum(x, src, dst):
    cols = []
    for j in range(x.shape[1] // 128):
        p0, p1 = _segsum128(x[:, j * 128:(j + 1) * 128], src, dst)
        cols.append(_combine(p0, p1))
    return jnp.concatenate(cols, axis=1) if len(cols) > 1 else cols[0]


GBLK = 64
GBLK1 = 16


def _zero_shared(s, zbuf, shared):
    zbuf[...] = jnp.zeros_like(zbuf)
    for i in range(STRIPE // 64):
        pltpu.sync_copy(zbuf, shared.at[pl.ds(s * STRIPE + i * 64, 64)])
    plsc.subcore_barrier()


def _writeback(c, s, shared, outp_hbm):
    plsc.subcore_barrier()
    pltpu.sync_copy(shared.at[pl.ds(s * STRIPE, STRIPE)],
                    outp_hbm.at[pl.ds(pl.multiple_of(c * SHR + s * STRIPE, 8),
                                      STRIPE)])


def _zero_lrow_tail(lrow, nblk):
    def z(e, _):
        for cc in range(1, 8):
            lrow[e, pl.ds(cc * 16, 16)] = jnp.zeros((16,), jnp.float32)
        return 0

    lax.fori_loop(0, nblk, z, 0)


def _gat1_logits(rs, rd, attv, padf, e):
    louts = []
    for h in range(4):
        acc = jnp.zeros((16,), jnp.float32)
        for cc in range(16):
            o = h * 256 + cc * 16
            z = rs[e, pl.ds(o, 16)] + rd[e, pl.ds(o, 16)]
            lr = jnp.maximum(z, 0.2 * z)
            acc = acc + lr * attv[pl.ds(o, 16)]
        pre = acc
        for sh in (1, 2, 4, 8):
            padf[pl.ds(16, 16)] = pre
            pre = pre + padf[pl.ds(16 - sh, 16)]
        louts.append(pre[15])
    return louts


def _gat1_mean_body(xl_hbm, xrp_hbm, src_hbm, dst_hbm, att_hbm, outp_hbm,
                    sbuf, dbuf, rs, rd, attv, lrow, padf, zbuf, shared,
                    sem, sem2):
    E2P = src_hbm.shape[0]
    epw = E2P // NW
    c = lax.axis_index("c")
    s = lax.axis_index("s")
    w = s * 2 + c
    iota = _iota16()
    _zero_shared(s, zbuf, shared)
    _zero_lrow_tail(lrow, GBLK1)
    padf[pl.ds(0, 16)] = jnp.zeros((16,), jnp.float32)
    pltpu.sync_copy(att_hbm, attv)
    base = w * epw

    def blk(b, _):
        off = pl.multiple_of(base + b * GBLK1, 8)
        pltpu.sync_copy(src_hbm.at[pl.ds(off, GBLK1)], sbuf)
        pltpu.sync_copy(dst_hbm.at[pl.ds(off, GBLK1)], dbuf)
        c1 = pltpu.async_copy(xl_hbm.at[sbuf], rs, sem)
        c2 = pltpu.async_copy(xrp_hbm.at[dbuf], rd, sem2)
        c1.wait()
        c2.wait()

        def edge(e, _):
            louts = _gat1_logits(rs, rd, attv, padf, e)
            row = jnp.where(iota == 4, 1.0, 0.0).astype(jnp.float32)
            for h in range(4):
                row = jnp.where(iota == h, louts[h], row)
            lrow[e, pl.ds(0, 16)] = row
            return 0

        lax.fori_loop(0, GBLK1, edge, 0)
        pltpu.async_copy(lrow, shared.at[dbuf], sem, add=True).wait()
        return 0

    lax.fori_loop(0, epw // GBLK1, blk, 0)
    _writeback(c, s, shared, outp_hbm)


def _gat1_wts_body(xl_hbm, xrp_hbm, src_hbm, dst_hbm, att_hbm, meanp_hbm,
                   outp_hbm, w_hbm,
                   sbuf, dbuf, rs, rd, mr, attv, lrow, wstage, padf, zbuf,
                   shared, sem, sem2, sem3):
    E2P = src_hbm.shape[0]
    epw = E2P // NW
    c = lax.axis_index("c")
    s = lax.axis_index("s")
    w = s * 2 + c
    iota = _iota16()
    _zero_shared(s, zbuf, shared)
    _zero_lrow_tail(lrow, GBLK1)
    padf[pl.ds(0, 16)] = jnp.zeros((16,), jnp.float32)
    pltpu.sync_copy(att_hbm, attv)
    base = w * epw
    zf = jnp.zeros((16,), jnp.float32)

    def blk(b, _):
        off = pl.multiple_of(base + b * GBLK1, 8)
        pltpu.sync_copy(src_hbm.at[pl.ds(off, GBLK1)], sbuf)
        pltpu.sync_copy(dst_hbm.at[pl.ds(off, GBLK1)], dbuf)
        c1 = pltpu.async_copy(xl_hbm.at[sbuf], rs, sem)
        c2 = pltpu.async_copy(xrp_hbm.at[dbuf], rd, sem2)
        c3 = pltpu.async_copy(meanp_hbm.at[dbuf], mr, sem3)
        c1.wait()
        c2.wait()
        c3.wait()

        def edge(e, _):
            louts = _gat1_logits(rs, rd, attv, padf, e)
            mrow = mr[e, pl.ds(0, 16)]
            row = zf
            for h in range(4):
                wv = jnp.exp(jnp.minimum(zf + (louts[h] - mrow[h]), 75.0))
                row = jnp.where(iota == h, wv, row)
            lrow[e, pl.ds(0, 16)] = row
            wstage[pl.ds(e * 16, 16)] = row
            return 0

        lax.fori_loop(0, GBLK1, edge, 0)
        pltpu.async_copy(lrow, shared.at[dbuf], sem, add=True).wait()
        pltpu.sync_copy(wstage,
                        w_hbm.at[pl.ds(pl.multiple_of(off * 16, 8), GBLK1 * 16)])
        return 0

    lax.fori_loop(0, epw // GBLK1, blk, 0)
    _writeback(c, s, shared, outp_hbm)


def _wseg_body(xcol_hbm, src_hbm, dst_hbm, w_hbm, outp_hbm,
               sbuf, dbuf, wbuf, rows, zbuf, shared, sem, *, hlane):
    E2P = src_hbm.shape[0]
    epw = E2P // NW
    c = lax.axis_index("c")
    s = lax.axis_index("s")
    w = s * 2 + c
    _zero_shared(s, zbuf, shared)
    base = w * epw

    def blk(b, _):
        off = pl.multiple_of(base + b * GBLK, 8)
        pltpu.sync_copy(src_hbm.at[pl.ds(off, GBLK)], sbuf)
        pltpu.sync_copy(dst_hbm.at[pl.ds(off, GBLK)], dbuf)
        pltpu.sync_copy(w_hbm.at[pl.ds(pl.multiple_of(off * 16, 8), GBLK * 16)],
                        wbuf)
        pltpu.async_copy(xcol_hbm.at[sbuf], rows, sem).wait()

        def edge(e, _):
            wv = wbuf[pl.ds(e * 16, 16)]
            ws = wv[hlane]
            for cc in range(8):
                rows[e, pl.ds(cc * 16, 16)] = rows[e, pl.ds(cc * 16, 16)] * ws
            return 0

        lax.fori_loop(0, GBLK, edge, 0)
        pltpu.async_copy(rows, shared.at[dbuf], sem, add=True).wait()
        return 0

    lax.fori_loop(0, epw // GBLK, blk, 0)
    _writeback(c, s, shared, outp_hbm)


def _gat2_body(xla_hbm, xrp_hbm, src_hbm, dst_hbm, att_hbm, meanp_hbm,
               outp_hbm, sbuf, dbuf, xa, xb, mr, attv, lrow, zbuf, shared,
               sem, sem2, sem3, *, mode):
    E2P = src_hbm.shape[0]
    epw = E2P // NW
    c = lax.axis_index("c")
    s = lax.axis_index("s")
    w = s * 2 + c
    iota = _iota16()
    _zero_shared(s, zbuf, shared)
    _zero_lrow_tail(lrow, GBLK)
    pltpu.sync_copy(att_hbm, attv)
    att2s = attv[...][0]
    base = w * epw
    zf = jnp.zeros((16,), jnp.float32)

    def blk(b, _):
        off = pl.multiple_of(base + b * GBLK, 8)
        pltpu.sync_copy(src_hbm.at[pl.ds(off, GBLK)], sbuf)
        pltpu.sync_copy(dst_hbm.at[pl.ds(off, GBLK)], dbuf)
        c1 = pltpu.async_copy(xla_hbm.at[sbuf], xa, sem)
        c2 = pltpu.async_copy(xrp_hbm.at[dbuf], xb, sem2)
        if mode == "num":
            pltpu.async_copy(meanp_hbm.at[dbuf], mr, sem3).wait()
        c1.wait()
        c2.wait()

        def edge(e, _):
            a0 = xa[e, pl.ds(0, 16)]
            b0 = xb[e, pl.ds(0, 16)]
            z = a0 + b0
            lr = jnp.maximum(z, 0.2 * z)
            lv = lr * att2s
            if mode == "mean":
                row = lv + jnp.where(iota == 1, 1.0, 0.0).astype(jnp.float32)
            else:
                m0 = mr[e, pl.ds(0, 16)]
                wv = jnp.exp(jnp.minimum(zf + (lv[0] - m0[0]), 75.0))
                row = jnp.where(iota == 0, wv * a0[0],
                                jnp.where(iota == 1, wv, zf)).astype(
                                    jnp.float32)
            lrow[e, pl.ds(0, 16)] = row
            return 0

        lax.fori_loop(0, GBLK, edge, 0)
        pltpu.async_copy(lrow, shared.at[dbuf], sem, add=True).wait()
        return 0

    lax.fori_loop(0, epw // GBLK, blk, 0)
    _writeback(c, s, shared, outp_hbm)


def _gat1_mean(xl, xrp, src, dst, attf):
    k = pl.kernel(
        _gat1_mean_body,
        mesh=_mesh,
        out_type=jax.ShapeDtypeStruct((2 * SHR, 128), jnp.float32),
        scratch_types=[pltpu.VMEM((GBLK1,), jnp.int32),
                       pltpu.VMEM((GBLK1,), jnp.int32),
                       pltpu.VMEM((GBLK1, 1024), jnp.float32),
                       pltpu.VMEM((GBLK1, 1024), jnp.float32),
                       pltpu.VMEM((1024,), jnp.float32),
                       pltpu.VMEM((GBLK1, 128), jnp.float32),
                       pltpu.VMEM((32,), jnp.float32),
                       pltpu.VMEM((64, 128), jnp.float32),
                       pltpu.VMEM_SHARED((SHR, 128), jnp.float32),
                       pltpu.SemaphoreType.DMA,
                       pltpu.SemaphoreType.DMA],
    )
    return k(xl, xrp, src, dst, attf)


def _gat1_wts(xl, xrp, src, dst, attf, meanp, e2p):
    k = pl.kernel(
        _gat1_wts_body,
        mesh=_mesh,
        out_type=[jax.ShapeDtypeStruct((2 * SHR, 128), jnp.float32),
                  jax.ShapeDtypeStruct((e2p * 16,), jnp.float32)],
        scratch_types=[pltpu.VMEM((GBLK1,), jnp.int32),
                       pltpu.VMEM((GBLK1,), jnp.int32),
                       pltpu.VMEM((GBLK1, 1024), jnp.float32),
                       pltpu.VMEM((GBLK1, 1024), jnp.float32),
                       pltpu.VMEM((GBLK1, 128), jnp.float32),
                       pltpu.VMEM((1024,), jnp.float32),
                       pltpu.VMEM((GBLK1, 128), jnp.float32),
                       pltpu.VMEM((GBLK1 * 16,), jnp.float32),
                       pltpu.VMEM((32,), jnp.float32),
                       pltpu.VMEM((64, 128), jnp.float32),
                       pltpu.VMEM_SHARED((SHR, 128), jnp.float32),
                       pltpu.SemaphoreType.DMA,
                       pltpu.SemaphoreType.DMA,
                       pltpu.SemaphoreType.DMA],
    )
    return k(xl, xrp, src, dst, attf, meanp)


def _wseg(xcol, src, dst, wts, hlane):
    k = pl.kernel(
        functools.partial(_wseg_body, hlane=hlane),
        mesh=_mesh,
        out_type=jax.ShapeDtypeStruct((2 * SHR, 128), jnp.float32),
        scratch_types=[pltpu.VMEM((GBLK,), jnp.int32),
                       pltpu.VMEM((GBLK,), jnp.int32),
                       pltpu.VMEM((GBLK * 16,), jnp.float32),
                       pltpu.VMEM((GBLK, 128), jnp.float32),
                       pltpu.VMEM((64, 128), jnp.float32),
                       pltpu.VMEM_SHARED((SHR, 128), jnp.float32),
                       pltpu.SemaphoreType.DMA],
    )
    return k(xcol, src, dst, wts)


def _gat2(xla, xrp, src, dst, att2f, meanp, mode):
    k = pl.kernel(
        functools.partial(_gat2_body, mode=mode),
        mesh=_mesh,
        out_type=jax.ShapeDtypeStruct((2 * SHR, 128), jnp.float32),
        scratch_types=[pltpu.VMEM((GBLK,), jnp.int32),
                       pltpu.VMEM((GBLK,), jnp.int32),
                       pltpu.VMEM((GBLK, 128), jnp.float32),
                       pltpu.VMEM((GBLK, 128), jnp.float32),
                       pltpu.VMEM((GBLK, 128), jnp.float32),
                       pltpu.VMEM((16,), jnp.float32),
                       pltpu.VMEM((GBLK, 128), jnp.float32),
                       pltpu.VMEM((64, 128), jnp.float32),
                       pltpu.VMEM_SHARED((SHR, 128), jnp.float32),
                       pltpu.SemaphoreType.DMA,
                       pltpu.SemaphoreType.DMA,
                       pltpu.SemaphoreType.DMA],
    )
    return k(xla, xrp, src, dst, att2f, meanp)


def _mm_stats_body(x_ref, agg_ref, w_ref, b_ref, h_ref, s1_ref, s2_ref):
    i = pl.program_id(0)
    u = x_ref[...] + agg_ref[...]
    h = jnp.dot(u, w_ref[...], preferred_element_type=jnp.float32) + b_ref[...]
    h_ref[...] = h

    @pl.when(i == 0)
    def _():
        s1_ref[...] = jnp.zeros_like(s1_ref)
        s2_ref[...] = jnp.zeros_like(s2_ref)

    s1_ref[...] += jnp.sum(h, axis=0, keepdims=True)
    s2_ref[...] += jnp.sum(h * h, axis=0, keepdims=True)


def _mm_stats(x, agg, w, b):
    n, k = x.shape
    c = w.shape[1]
    return pl.pallas_call(
        _mm_stats_body,
        grid=(n // BR,),
        in_specs=[
            pl.BlockSpec((BR, k), lambda i: (i, 0)),
            pl.BlockSpec((BR, k), lambda i: (i, 0)),
            pl.BlockSpec((k, c), lambda i: (0, 0)),
            pl.BlockSpec((1, c), lambda i: (0, 0)),
        ],
        out_specs=[
            pl.BlockSpec((BR, c), lambda i: (i, 0)),
            pl.BlockSpec((1, c), lambda i: (0, 0)),
            pl.BlockSpec((1, c), lambda i: (0, 0)),
        ],
        out_shape=[
            jax.ShapeDtypeStruct((n, c), jnp.float32),
            jax.ShapeDtypeStruct((1, c), jnp.float32),
            jax.ShapeDtypeStruct((1, c), jnp.float32),
        ],
    )(x, agg, w, b.reshape(1, c))


def _stats_body(s1_ref, s2_ref, g_ref, be_ref, a_ref, sh_ref):
    mean = s1_ref[...] * (1.0 / N)
    var = s2_ref[...] * (1.0 / N) - mean * mean
    a = g_ref[...] * jax.lax.rsqrt(var + 1e-5)
    a_ref[...] = a
    sh_ref[...] = be_ref[...] - mean * a


def _bn_scale(s1, s2, g, be):
    c = s1.shape[1]
    return pl.pallas_call(
        _stats_body,
        out_shape=[jax.ShapeDtypeStruct((1, c), jnp.float32),
                   jax.ShapeDtypeStruct((1, c), jnp.float32)],
    )(s1, s2, g.reshape(1, c), be.reshape(1, c))


def _bn_mm_body(h_ref, a_ref, sh_ref, w_ref, b_ref, o_ref, *, relu_out):
    t = jnp.maximum(h_ref[...] * a_ref[...] + sh_ref[...], 0.0)
    o = jnp.dot(t, w_ref[...], preferred_element_type=jnp.float32) + b_ref[...]
    if relu_out:
        o = jnp.maximum(o, 0.0)
    o_ref[...] = o


def _bn_mm(h, a, sh, w, b, relu_out):
    n, k = h.shape
    c = w.shape[1]
    return pl.pallas_call(
        functools.partial(_bn_mm_body, relu_out=relu_out),
        grid=(n // BR,),
        in_specs=[
            pl.BlockSpec((BR, k), lambda i: (i, 0)),
            pl.BlockSpec((1, k), lambda i: (0, 0)),
            pl.BlockSpec((1, k), lambda i: (0, 0)),
            pl.BlockSpec((k, c), lambda i: (0, 0)),
            pl.BlockSpec((1, c), lambda i: (0, 0)),
        ],
        out_specs=pl.BlockSpec((BR, c), lambda i: (i, 0)),
        out_shape=jax.ShapeDtypeStruct((n, c), jnp.float32),
    )(h, a, sh, w, b.reshape(1, c))


def _dual_mm_body(x_ref, wl_ref, bl_ref, wr_ref, br_ref, l_ref, r_ref):
    x = x_ref[...]
    l_ref[...] = jnp.dot(x, wl_ref[...], preferred_element_type=jnp.float32) + bl_ref[...]
    r_ref[...] = jnp.dot(x, wr_ref[...], preferred_element_type=jnp.float32) + br_ref[...]


def _dual_mm(x, wl, bl, wr, br):
    n, k = x.shape
    c = wl.shape[1]
    return pl.pallas_call(
        _dual_mm_body,
        grid=(n // BR,),
        in_specs=[
            pl.BlockSpec((BR, k), lambda i: (i, 0)),
            pl.BlockSpec((k, c), lambda i: (0, 0)),
            pl.BlockSpec((1, c), lambda i: (0, 0)),
            pl.BlockSpec((k, c), lambda i: (0, 0)),
            pl.BlockSpec((1, c), lambda i: (0, 0)),
        ],
        out_specs=[pl.BlockSpec((BR, c), lambda i: (i, 0)),
                   pl.BlockSpec((BR, c), lambda i: (i, 0))],
        out_shape=[jax.ShapeDtypeStruct((n, c), jnp.float32),
                   jax.ShapeDtypeStruct((n, c), jnp.float32)],
    )(x, wl, bl.reshape(1, c), wr, br.reshape(1, c))


def _final_body(flat_ref, wd_ref, bd_ref, out_ref):
    i = pl.program_id(0)

    @pl.when(i == 0)
    def _():
        out_ref[...] = jnp.zeros_like(out_ref)

    out_ref[...] += jnp.sum(flat_ref[...] * wd_ref[...]).reshape(1, 1)

    @pl.when(i == pl.num_programs(0) - 1)
    def _():
        out_ref[...] = jax.nn.sigmoid(out_ref[...] + bd_ref[...])


def _final_dot(flat, wd, bd):
    return pl.pallas_call(
        _final_body,
        grid=(N // BR,),
        in_specs=[pl.BlockSpec((BR, L), lambda i: (i, 0)),
                  pl.BlockSpec((BR, L), lambda i: (i, 0)),
                  pl.BlockSpec((1, 1), lambda i: (0, 0))],
        out_specs=pl.BlockSpec((1, 1), lambda i: (0, 0)),
        out_shape=jax.ShapeDtypeStruct((1, 1), jnp.float32),
    )(flat, wd, bd.reshape(1, 1))


def _mean1_body(p0_ref, p1_ref, o_ref):
    ps = p0_ref[...] + p1_ref[...]
    cnt = jnp.maximum(ps[:, 4:5], 1.0)
    m = ps[:, 0:4] / cnt
    o_ref[...] = jnp.concatenate(
        [m, jnp.zeros((m.shape[0], 124), jnp.float32)], axis=1)


def _mean1(p0, p1):
    return pl.pallas_call(
        _mean1_body,
        grid=(N // BR,),
        in_specs=[pl.BlockSpec((BR, 128), lambda i: (i, 0)),
                  pl.BlockSpec((BR, 128), lambda i: (i, 0))],
        out_specs=pl.BlockSpec((BR, 128), lambda i: (i, 0)),
        out_shape=jax.ShapeDtypeStruct((N, 128), jnp.float32),
    )(p0, p1)


def _gat1_norm_body(n_ref, d0_ref, d1_ref, b_ref, o_ref):
    den = (d0_ref[...] + d1_ref[...])[:, 0:4] + 1e-16
    scale = jnp.repeat(1.0 / den, 256, axis=1)
    o_ref[...] = n_ref[...] * scale + b_ref[...]


def _gat1_norm(num, d0, d1, bias1):
    return pl.pallas_call(
        _gat1_norm_body,
        grid=(N // BR,),
        in_specs=[pl.BlockSpec((BR, 1024), lambda i: (i, 0)),
                  pl.BlockSpec((BR, 128), lambda i: (i, 0)),
                  pl.BlockSpec((BR, 128), lambda i: (i, 0)),
                  pl.BlockSpec((1, 1024), lambda i: (0, 0))],
        out_specs=pl.BlockSpec((BR, 1024), lambda i: (i, 0)),
        out_shape=jax.ShapeDtypeStruct((N, 1024), jnp.float32),
    )(num, d0, d1, bias1.reshape(1, 1024))


def _gat2_proj_body(x_ref, wl_ref, bl_ref, wr_ref, br_ref, la_ref, ra_ref):
    x = x_ref[...]
    z = jnp.zeros((x.shape[0], 127), jnp.float32)
    xl2 = jnp.dot(x, wl_ref[...], preferred_element_type=jnp.float32) + bl_ref[...]
    xr2 = jnp.dot(x, wr_ref[...], preferred_element_type=jnp.float32) + br_ref[...]
    la_ref[...] = jnp.concatenate([xl2, z], axis=1)
    ra_ref[...] = jnp.concatenate([xr2, z], axis=1)


def _gat2_proj(r1, wl2, bl2, wr2, br2):
    return pl.pallas_call(
        _gat2_proj_body,
        grid=(N // BR,),
        in_specs=[pl.BlockSpec((BR, 1024), lambda i: (i, 0)),
                  pl.BlockSpec((1024, 1), lambda i: (0, 0)),
                  pl.BlockSpec((1, 1), lambda i: (0, 0)),
                  pl.BlockSpec((1024, 1), lambda i: (0, 0)),
                  pl.BlockSpec((1, 1), lambda i: (0, 0))],
        out_specs=[pl.BlockSpec((BR, 128), lambda i: (i, 0)),
                   pl.BlockSpec((BR, 128), lambda i: (i, 0))],
        out_shape=[jax.ShapeDtypeStruct((N, 128), jnp.float32),
                   jax.ShapeDtypeStruct((N, 128), jnp.float32)],
    )(r1, wl2, bl2.reshape(1, 1), wr2, br2.reshape(1, 1))


def _gat2_mean_body(p0_ref, p1_ref, o_ref):
    ps = p0_ref[...] + p1_ref[...]
    m = ps[:, 0:1] / jnp.maximum(ps[:, 1:2], 1.0)
    o_ref[...] = jnp.concatenate(
        [m, jnp.zeros((m.shape[0], 127), jnp.float32)], axis=1)


def _gat2_mean(p0, p1):
    return pl.pallas_call(
        _gat2_mean_body,
        grid=(N // BR,),
        in_specs=[pl.BlockSpec((BR, 128), lambda i: (i, 0)),
                  pl.BlockSpec((BR, 128), lambda i: (i, 0))],
        out_specs=pl.BlockSpec((BR, 128), lambda i: (i, 0)),
        out_shape=jax.ShapeDtypeStruct((N, 128), jnp.float32),
    )(p0, p1)


def _gat2_score_body(p0_ref, p1_ref, b_ref, o_ref):
    ps = p0_ref[...] + p1_ref[...]
    sc = ps[:, 0:1] / (ps[:, 1:2] + 1e-16) + b_ref[...]
    o_ref[...] = jnp.concatenate(
        [sc, jnp.zeros((sc.shape[0], 127), jnp.float32)], axis=1)


def _gat2_score(p0, p1, bias2):
    return pl.pallas_call(
        _gat2_score_body,
        grid=(N // BR,),
        in_specs=[pl.BlockSpec((BR, 128), lambda i: (i, 0)),
                  pl.BlockSpec((BR, 128), lambda i: (i, 0)),
                  pl.BlockSpec((1, 1), lambda i: (0, 0))],
        out_specs=pl.BlockSpec((BR, 128), lambda i: (i, 0)),
        out_shape=jax.ShapeDtypeStruct((N, 128), jnp.float32),
    )(p0, p1, bias2.reshape(1, 1))


def _gin_conv(x, agg, W1, b1, g, be, W2, b2, relu_out):
    h, s1, s2 = _mm_stats(x, agg, W1, b1)
    a, sh = _bn_scale(s1, s2, g, be)
    return _bn_mm(h, a, sh, W2, b2, relu_out)


def kernel(eeg_nodes, eeg_idx, W11, b11, g1, be1, W12, b12, W21, b21, g2, be2, W22, b22,
           Wl1, bl1, Wr1, br1, att1, bias1, Wl2, bl2, Wr2, br2, att2, bias2, Wd, bd):
    src = eeg_idx[0].astype(jnp.int32)
    dst = eeg_idx[1].astype(jnp.int32)
    E = src.shape[0]
    loop = jnp.arange(N, dtype=jnp.int32)
    E2 = E + N
    E2P = -(-E2 // (NW * 64)) * (NW * 64)
    s2p = jnp.concatenate([src, loop, jnp.zeros((E2P - E2,), jnp.int32)])
    d2p = jnp.concatenate([dst, loop,
                           jnp.full((E2P - E2,), SHR - 1, jnp.int32)])

    agg1 = _segsum(eeg_nodes, src, dst)
    h = _gin_conv(eeg_nodes, agg1, W11, b11, g1, be1, W12, b12, relu_out=True)
    agg2 = _segsum(h, src, dst)
    h = _gin_conv(h, agg2, W21, b21, g2, be2, W22, b22, relu_out=False)

    xl1, xr1 = _dual_mm(h, Wl1, bl1, Wr1, br1)
    xr1p = jnp.pad(xr1, ((0, SHR - N), (0, 0)))
    attf = att1.reshape(1024)
    mp = _gat1_mean(xl1, xr1p, s2p, d2p, attf)
    meanp = jnp.pad(_mean1(mp[:N], mp[SHR:SHR + N]), ((0, SHR - N), (0, 0)))
    wp, wts = _gat1_wts(xl1, xr1p, s2p, d2p, attf, meanp, E2P)
    cols = []
    for j in range(8):
        pj = _wseg(xl1[:, j * 128:(j + 1) * 128], s2p, d2p, wts, j // 2)
        cols.append(_combine(pj[:N], pj[SHR:SHR + N]))
    num = jnp.concatenate(cols, axis=1)
    r1 = _gat1_norm(num, wp[:N], wp[SHR:SHR + N], bias1)

    xla, xra = _gat2_proj(r1, Wl2, bl2, Wr2, br2)
    xrap = jnp.pad(xra, ((0, SHR - N), (0, 0)))
    att2f = jnp.pad(att2.reshape(1), (0, 15))
    m2p = _gat2(xla, xrap, s2p, d2p, att2f, xrap, mode="mean")
    mean2p = jnp.pad(_gat2_mean(m2p[:N], m2p[SHR:SHR + N]),
                     ((0, SHR - N), (0, 0)))
    q = _gat2(xla, xrap, s2p, d2p, att2f, mean2p, mode="num")
    region_scores = _gat2_score(q[:N], q[SHR:SHR + N], bias2)[:, :1]

    dementia_pred = _final_dot(h, Wd.reshape(N, L), bd)
    return (dementia_pred, region_scores)


# speedup vs baseline: 4.7333x; 1.3249x over previous
import functools
import jax
import jax.numpy as jnp
from jax import lax
from jax.experimental import pallas as pl
from jax.experimental.pallas import tpu as pltpu
from jax.experimental.pallas import tpu_sc as plsc

N = 10000
T = 256
HID = 512
L = 256

BR = 1000

NW = 32
SHR = 10240
STRIPE = SHR // 16
BLK = 200

_mesh = plsc.VectorSubcoreMesh(core_axis_name="c", subcore_axis_name="s")


def _iota16():
    return lax.iota(jnp.int32, 16)


def _segsum_body(x_hbm, src_hbm, dst_hbm, outp_hbm,
                 idxbuf, dstbuf, rows, zbuf, shared, sem):
    E = src_hbm.shape[0]
    epw = E // NW
    c = lax.axis_index("c")
    s = lax.axis_index("s")
    w = s * 2 + c
    zbuf[...] = jnp.zeros_like(zbuf)
    for i in range(STRIPE // 64):
        pltpu.sync_copy(zbuf, shared.at[pl.ds(s * STRIPE + i * 64, 64)])
    plsc.subcore_barrier()
    base = w * epw

    def blk(b, _):
        off = pl.multiple_of(base + b * BLK, 8)
        pltpu.sync_copy(src_hbm.at[pl.ds(off, BLK)], idxbuf)
        pltpu.sync_copy(dst_hbm.at[pl.ds(off, BLK)], dstbuf)
        pltpu.async_copy(x_hbm.at[idxbuf], rows, sem).wait()
        pltpu.async_copy(rows, shared.at[dstbuf], sem, add=True).wait()
        return 0

    lax.fori_loop(0, epw // BLK, blk, 0)
    plsc.subcore_barrier()
    pltpu.sync_copy(shared.at[pl.ds(s * STRIPE, STRIPE)],
                    outp_hbm.at[pl.ds(pl.multiple_of(c * SHR + s * STRIPE, 8),
                                      STRIPE)])


def _segsum128(x, src, dst):
    k = pl.kernel(
        _segsum_body,
        mesh=_mesh,
        out_type=jax.ShapeDtypeStruct((2 * SHR, 128), jnp.float32),
        scratch_types=[pltpu.VMEM((BLK,), jnp.int32),
                       pltpu.VMEM((BLK,), jnp.int32),
                       pltpu.VMEM((BLK, 128), jnp.float32),
                       pltpu.VMEM((64, 128), jnp.float32),
                       pltpu.VMEM_SHARED((SHR, 128), jnp.float32),
                       pltpu.SemaphoreType.DMA],
    )
    outp = k(x, src, dst)
    return outp[:N], outp[SHR:SHR + N]


def _combine_body(a_ref, b_ref, o_ref):
    o_ref[...] = a_ref[...] + b_ref[...]


def _combine(a, b):
    n, w_ = a.shape
    return pl.pallas_call(
        _combine_body,
        grid=(n // BR,),
        in_specs=[pl.BlockSpec((BR, w_), lambda i: (i, 0)),
                  pl.BlockSpec((BR, w_), lambda i: (i, 0))],
        out_specs=pl.BlockSpec((BR, w_), lambda i: (i, 0)),
        out_shape=jax.ShapeDtypeStruct((n, w_), jnp.float32),
    )(a, b)


def _segsum(x, src, dst):
    cols = []
    for j in range(x.shape[1] // 128):
        p0, p1 = _segsum128(x[:, j * 128:(j + 1) * 128], src, dst)
        cols.append(_combine(p0, p1))
    return jnp.concatenate(cols, axis=1) if len(cols) > 1 else cols[0]


GBLK = 64
GBLK1 = 16


def _zero_shared(s, zbuf, shared):
    zbuf[...] = jnp.zeros_like(zbuf)
    for i in range(STRIPE // 64):
        pltpu.sync_copy(zbuf, shared.at[pl.ds(s * STRIPE + i * 64, 64)])
    plsc.subcore_barrier()


def _writeback(c, s, shared, outp_hbm):
    plsc.subcore_barrier()
    pltpu.sync_copy(shared.at[pl.ds(s * STRIPE, STRIPE)],
                    outp_hbm.at[pl.ds(pl.multiple_of(c * SHR + s * STRIPE, 8),
                                      STRIPE)])


def _zero_lrow_tail(lrow, nblk):
    def z(e, _):
        for cc in range(1, 8):
            lrow[e, pl.ds(cc * 16, 16)] = jnp.zeros((16,), jnp.float32)
        return 0

    lax.fori_loop(0, nblk, z, 0)


def _gat1_logits(rs, rd, attv, padf, e):
    louts = []
    for h in range(4):
        acc = jnp.zeros((16,), jnp.float32)
        for cc in range(16):
            o = h * 256 + cc * 16
            z = rs[e, pl.ds(o, 16)] + rd[e, pl.ds(o, 16)]
            lr = jnp.maximum(z, 0.2 * z)
            acc = acc + lr * attv[pl.ds(o, 16)]
        pre = acc
        for sh in (1, 2, 4, 8):
            padf[pl.ds(16, 16)] = pre
            pre = pre + padf[pl.ds(16 - sh, 16)]
        louts.append(pre[15])
    return louts


def _gat1_mean_body(xl_hbm, xrp_hbm, src_hbm, dst_hbm, att_hbm, outp_hbm,
                    sbuf, dbuf, rs, rd, attv, lrow, padf, zbuf, shared,
                    sem, sem2):
    E2P = src_hbm.shape[0]
    epw = E2P // NW
    c = lax.axis_index("c")
    s = lax.axis_index("s")
    w = s * 2 + c
    iota = _iota16()
    _zero_shared(s, zbuf, shared)
    _zero_lrow_tail(lrow, GBLK1)
    padf[pl.ds(0, 16)] = jnp.zeros((16,), jnp.float32)
    pltpu.sync_copy(att_hbm, attv)
    base = w * epw

    def blk(b, _):
        off = pl.multiple_of(base + b * GBLK1, 8)
        pltpu.sync_copy(src_hbm.at[pl.ds(off, GBLK1)], sbuf)
        pltpu.sync_copy(dst_hbm.at[pl.ds(off, GBLK1)], dbuf)
        c1 = pltpu.async_copy(xl_hbm.at[sbuf], rs, sem)
        c2 = pltpu.async_copy(xrp_hbm.at[dbuf], rd, sem2)
        c1.wait()
        c2.wait()

        def edge(e, _):
            louts = _gat1_logits(rs, rd, attv, padf, e)
            row = jnp.where(iota == 4, 1.0, 0.0).astype(jnp.float32)
            for h in range(4):
                row = jnp.where(iota == h, louts[h], row)
            lrow[e, pl.ds(0, 16)] = row
            return 0

        lax.fori_loop(0, GBLK1, edge, 0)
        pltpu.async_copy(lrow, shared.at[dbuf], sem, add=True).wait()
        return 0

    lax.fori_loop(0, epw // GBLK1, blk, 0)
    _writeback(c, s, shared, outp_hbm)


def _gat1_wts_body(xl_hbm, xrp_hbm, src_hbm, dst_hbm, att_hbm, meanp_hbm,
                   outp_hbm, w_hbm,
                   sbuf, dbuf, rs, rd, mr, attv, lrow, wstage, padf, zbuf,
                   shared, sem, sem2, sem3):
    E2P = src_hbm.shape[0]
    epw = E2P // NW
    c = lax.axis_index("c")
    s = lax.axis_index("s")
    w = s * 2 + c
    iota = _iota16()
    _zero_shared(s, zbuf, shared)
    _zero_lrow_tail(lrow, GBLK1)
    padf[pl.ds(0, 16)] = jnp.zeros((16,), jnp.float32)
    pltpu.sync_copy(att_hbm, attv)
    base = w * epw
    zf = jnp.zeros((16,), jnp.float32)

    def blk(b, _):
        off = pl.multiple_of(base + b * GBLK1, 8)
        pltpu.sync_copy(src_hbm.at[pl.ds(off, GBLK1)], sbuf)
        pltpu.sync_copy(dst_hbm.at[pl.ds(off, GBLK1)], dbuf)
        c1 = pltpu.async_copy(xl_hbm.at[sbuf], rs, sem)
        c2 = pltpu.async_copy(xrp_hbm.at[dbuf], rd, sem2)
        c3 = pltpu.async_copy(meanp_hbm.at[dbuf], mr, sem3)
        c1.wait()
        c2.wait()
        c3.wait()

        def edge(e, _):
            louts = _gat1_logits(rs, rd, attv, padf, e)
            mrow = mr[e, pl.ds(0, 16)]
            row = zf
            for h in range(4):
                wv = jnp.exp(jnp.minimum(zf + (louts[h] - mrow[h]), 75.0))
                row = jnp.where(iota == h, wv, row)
            lrow[e, pl.ds(0, 16)] = row
            wstage[pl.ds(e * 16, 16)] = row
            return 0

        lax.fori_loop(0, GBLK1, edge, 0)
        pltpu.async_copy(lrow, shared.at[dbuf], sem, add=True).wait()
        pltpu.sync_copy(wstage,
                        w_hbm.at[pl.ds(pl.multiple_of(off * 16, 8), GBLK1 * 16)])
        return 0

    lax.fori_loop(0, epw // GBLK1, blk, 0)
    _writeback(c, s, shared, outp_hbm)


def _wseg_body(xcol_hbm, src_hbm, dst_hbm, w_hbm, outp_hbm,
               sbuf, dbuf, wbuf, rows, zbuf, shared, sem, *, hlane):
    E2P = src_hbm.shape[0]
    epw = E2P // NW
    c = lax.axis_index("c")
    s = lax.axis_index("s")
    w = s * 2 + c
    _zero_shared(s, zbuf, shared)
    base = w * epw

    def blk(b, _):
        off = pl.multiple_of(base + b * GBLK, 8)
        pltpu.sync_copy(src_hbm.at[pl.ds(off, GBLK)], sbuf)
        pltpu.sync_copy(dst_hbm.at[pl.ds(off, GBLK)], dbuf)
        pltpu.sync_copy(w_hbm.at[pl.ds(pl.multiple_of(off * 16, 8), GBLK * 16)],
                        wbuf)
        pltpu.async_copy(xcol_hbm.at[sbuf], rows, sem).wait()

        def edge(e, _):
            wv = wbuf[pl.ds(e * 16, 16)]
            ws = wv[hlane]
            for cc in range(8):
                rows[e, pl.ds(cc * 16, 16)] = rows[e, pl.ds(cc * 16, 16)] * ws
            return 0

        lax.fori_loop(0, GBLK, edge, 0)
        pltpu.async_copy(rows, shared.at[dbuf], sem, add=True).wait()
        return 0

    lax.fori_loop(0, epw // GBLK, blk, 0)
    _writeback(c, s, shared, outp_hbm)


def _gat2_body(xla_hbm, xrp_hbm, src_hbm, dst_hbm, att_hbm, meanp_hbm,
               outp_hbm, sbuf, dbuf, xa, xb, mr, attv, lrow, zbuf, shared,
               sem, sem2, sem3, *, mode):
    E2P = src_hbm.shape[0]
    epw = E2P // NW
    c = lax.axis_index("c")
    s = lax.axis_index("s")
    w = s * 2 + c
    iota = _iota16()
    _zero_shared(s, zbuf, shared)
    _zero_lrow_tail(lrow, GBLK)
    pltpu.sync_copy(att_hbm, attv)
    att2s = attv[...][0]
    base = w * epw
    zf = jnp.zeros((16,), jnp.float32)

    def blk(b, _):
        off = pl.multiple_of(base + b * GBLK, 8)
        pltpu.sync_copy(src_hbm.at[pl.ds(off, GBLK)], sbuf)
        pltpu.sync_copy(dst_hbm.at[pl.ds(off, GBLK)], dbuf)
        c1 = pltpu.async_copy(xla_hbm.at[sbuf], xa, sem)
        c2 = pltpu.async_copy(xrp_hbm.at[dbuf], xb, sem2)
        if mode == "num":
            pltpu.async_copy(meanp_hbm.at[dbuf], mr, sem3).wait()
        c1.wait()
        c2.wait()

        def edge(e, _):
            a0 = xa[e, pl.ds(0, 16)]
            b0 = xb[e, pl.ds(0, 16)]
            z = a0 + b0
            lr = jnp.maximum(z, 0.2 * z)
            lv = lr * att2s
            if mode == "mean":
                row = lv + jnp.where(iota == 1, 1.0, 0.0).astype(jnp.float32)
            else:
                m0 = mr[e, pl.ds(0, 16)]
                wv = jnp.exp(jnp.minimum(zf + (lv[0] - m0[0]), 75.0))
                row = jnp.where(iota == 0, wv * a0[0],
                                jnp.where(iota == 1, wv, zf)).astype(
                                    jnp.float32)
            lrow[e, pl.ds(0, 16)] = row
            return 0

        lax.fori_loop(0, GBLK, edge, 0)
        pltpu.async_copy(lrow, shared.at[dbuf], sem, add=True).wait()
        return 0

    lax.fori_loop(0, epw // GBLK, blk, 0)
    _writeback(c, s, shared, outp_hbm)


def _gat1_mean(xl, xrp, src, dst, attf):
    k = pl.kernel(
        _gat1_mean_body,
        mesh=_mesh,
        out_type=jax.ShapeDtypeStruct((2 * SHR, 128), jnp.float32),
        scratch_types=[pltpu.VMEM((GBLK1,), jnp.int32),
                       pltpu.VMEM((GBLK1,), jnp.int32),
                       pltpu.VMEM((GBLK1, 1024), jnp.float32),
                       pltpu.VMEM((GBLK1, 1024), jnp.float32),
                       pltpu.VMEM((1024,), jnp.float32),
                       pltpu.VMEM((GBLK1, 128), jnp.float32),
                       pltpu.VMEM((32,), jnp.float32),
                       pltpu.VMEM((64, 128), jnp.float32),
                       pltpu.VMEM_SHARED((SHR, 128), jnp.float32),
                       pltpu.SemaphoreType.DMA,
                       pltpu.SemaphoreType.DMA],
    )
    return k(xl, xrp, src, dst, attf)


def _gat1_wts(xl, xrp, src, dst, attf, meanp, e2p):
    k = pl.kernel(
        _gat1_wts_body,
        mesh=_mesh,
        out_type=[jax.ShapeDtypeStruct((2 * SHR, 128), jnp.float32),
                  jax.ShapeDtypeStruct((e2p * 16,), jnp.float32)],
        scratch_types=[pltpu.VMEM((GBLK1,), jnp.int32),
                       pltpu.VMEM((GBLK1,), jnp.int32),
                       pltpu.VMEM((GBLK1, 1024), jnp.float32),
                       pltpu.VMEM((GBLK1, 1024), jnp.float32),
                       pltpu.VMEM((GBLK1, 128), jnp.float32),
                       pltpu.VMEM((1024,), jnp.float32),
                       pltpu.VMEM((GBLK1, 128), jnp.float32),
                       pltpu.VMEM((GBLK1 * 16,), jnp.float32),
                       pltpu.VMEM((32,), jnp.float32),
                       pltpu.VMEM((64, 128), jnp.float32),
                       pltpu.VMEM_SHARED((SHR, 128), jnp.float32),
                       pltpu.SemaphoreType.DMA,
                       pltpu.SemaphoreType.DMA,
                       pltpu.SemaphoreType.DMA],
    )
    return k(xl, xrp, src, dst, attf, meanp)


def _wseg(xcol, src, dst, wts, hlane):
    k = pl.kernel(
        functools.partial(_wseg_body, hlane=hlane),
        mesh=_mesh,
        out_type=jax.ShapeDtypeStruct((2 * SHR, 128), jnp.float32),
        scratch_types=[pltpu.VMEM((GBLK,), jnp.int32),
                       pltpu.VMEM((GBLK,), jnp.int32),
                       pltpu.VMEM((GBLK * 16,), jnp.float32),
                       pltpu.VMEM((GBLK, 128), jnp.float32),
                       pltpu.VMEM((64, 128), jnp.float32),
                       pltpu.VMEM_SHARED((SHR, 128), jnp.float32),
                       pltpu.SemaphoreType.DMA],
    )
    return k(xcol, src, dst, wts)


def _gat2(xla, xrp, src, dst, att2f, meanp, mode):
    k = pl.kernel(
        functools.partial(_gat2_body, mode=mode),
        mesh=_mesh,
        out_type=jax.ShapeDtypeStruct((2 * SHR, 128), jnp.float32),
        scratch_types=[pltpu.VMEM((GBLK,), jnp.int32),
                       pltpu.VMEM((GBLK,), jnp.int32),
                       pltpu.VMEM((GBLK, 128), jnp.float32),
                       pltpu.VMEM((GBLK, 128), jnp.float32),
                       pltpu.VMEM((GBLK, 128), jnp.float32),
                       pltpu.VMEM((16,), jnp.float32),
                       pltpu.VMEM((GBLK, 128), jnp.float32),
                       pltpu.VMEM((64, 128), jnp.float32),
                       pltpu.VMEM_SHARED((SHR, 128), jnp.float32),
                       pltpu.SemaphoreType.DMA,
                       pltpu.SemaphoreType.DMA,
                       pltpu.SemaphoreType.DMA],
    )
    return k(xla, xrp, src, dst, att2f, meanp)


def _mm_stats_body(x_ref, agg_ref, w_ref, b_ref, h_ref, s1_ref, s2_ref):
    i = pl.program_id(0)
    u = x_ref[...] + agg_ref[...]
    h = jnp.dot(u, w_ref[...], preferred_element_type=jnp.float32) + b_ref[...]
    h_ref[...] = h

    @pl.when(i == 0)
    def _():
        s1_ref[...] = jnp.zeros_like(s1_ref)
        s2_ref[...] = jnp.zeros_like(s2_ref)

    s1_ref[...] += jnp.sum(h, axis=0, keepdims=True)
    s2_ref[...] += jnp.sum(h * h, axis=0, keepdims=True)


def _mm_stats(x, agg, w, b):
    n, k = x.shape
    c = w.shape[1]
    return pl.pallas_call(
        _mm_stats_body,
        grid=(n // BR,),
        in_specs=[
            pl.BlockSpec((BR, k), lambda i: (i, 0)),
            pl.BlockSpec((BR, k), lambda i: (i, 0)),
            pl.BlockSpec((k, c), lambda i: (0, 0)),
            pl.BlockSpec((1, c), lambda i: (0, 0)),
        ],
        out_specs=[
            pl.BlockSpec((BR, c), lambda i: (i, 0)),
            pl.BlockSpec((1, c), lambda i: (0, 0)),
            pl.BlockSpec((1, c), lambda i: (0, 0)),
        ],
        out_shape=[
            jax.ShapeDtypeStruct((n, c), jnp.float32),
            jax.ShapeDtypeStruct((1, c), jnp.float32),
            jax.ShapeDtypeStruct((1, c), jnp.float32),
        ],
    )(x, agg, w, b.reshape(1, c))


def _stats_body(s1_ref, s2_ref, g_ref, be_ref, a_ref, sh_ref):
    mean = s1_ref[...] * (1.0 / N)
    var = s2_ref[...] * (1.0 / N) - mean * mean
    a = g_ref[...] * jax.lax.rsqrt(var + 1e-5)
    a_ref[...] = a
    sh_ref[...] = be_ref[...] - mean * a


def _bn_scale(s1, s2, g, be):
    c = s1.shape[1]
    return pl.pallas_call(
        _stats_body,
        out_shape=[jax.ShapeDtypeStruct((1, c), jnp.float32),
                   jax.ShapeDtypeStruct((1, c), jnp.float32)],
    )(s1, s2, g.reshape(1, c), be.reshape(1, c))


def _bn_mm_body(h_ref, a_ref, sh_ref, w_ref, b_ref, o_ref, *, relu_out):
    t = jnp.maximum(h_ref[...] * a_ref[...] + sh_ref[...], 0.0)
    o = jnp.dot(t, w_ref[...], preferred_element_type=jnp.float32) + b_ref[...]
    if relu_out:
        o = jnp.maximum(o, 0.0)
    o_ref[...] = o


def _bn_mm(h, a, sh, w, b, relu_out):
    n, k = h.shape
    c = w.shape[1]
    return pl.pallas_call(
        functools.partial(_bn_mm_body, relu_out=relu_out),
        grid=(n // BR,),
        in_specs=[
            pl.BlockSpec((BR, k), lambda i: (i, 0)),
            pl.BlockSpec((1, k), lambda i: (0, 0)),
            pl.BlockSpec((1, k), lambda i: (0, 0)),
            pl.BlockSpec((k, c), lambda i: (0, 0)),
            pl.BlockSpec((1, c), lambda i: (0, 0)),
        ],
        out_specs=pl.BlockSpec((BR, c), lambda i: (i, 0)),
        out_shape=jax.ShapeDtypeStruct((n, c), jnp.float32),
    )(h, a, sh, w, b.reshape(1, c))


def _dual_mm_body(x_ref, wl_ref, bl_ref, wr_ref, br_ref, l_ref, r_ref):
    x = x_ref[...]
    l_ref[...] = jnp.dot(x, wl_ref[...], preferred_element_type=jnp.float32) + bl_ref[...]
    r_ref[...] = jnp.dot(x, wr_ref[...], preferred_element_type=jnp.float32) + br_ref[...]


def _dual_mm(x, wl, bl, wr, br):
    n, k = x.shape
    c = wl.shape[1]
    return pl.pallas_call(
        _dual_mm_body,
        grid=(n // BR,),
        in_specs=[
            pl.BlockSpec((BR, k), lambda i: (i, 0)),
            pl.BlockSpec((k, c), lambda i: (0, 0)),
            pl.BlockSpec((1, c), lambda i: (0, 0)),
            pl.BlockSpec((k, c), lambda i: (0, 0)),
            pl.BlockSpec((1, c), lambda i: (0, 0)),
        ],
        out_specs=[pl.BlockSpec((BR, c), lambda i: (i, 0)),
                   pl.BlockSpec((BR, c), lambda i: (i, 0))],
        out_shape=[jax.ShapeDtypeStruct((n, c), jnp.float32),
                   jax.ShapeDtypeStruct((n, c), jnp.float32)],
    )(x, wl, bl.reshape(1, c), wr, br.reshape(1, c))


def _final_body(flat_ref, wd_ref, bd_ref, out_ref):
    i = pl.program_id(0)

    @pl.when(i == 0)
    def _():
        out_ref[...] = jnp.zeros_like(out_ref)

    out_ref[...] += jnp.sum(flat_ref[...] * wd_ref[...]).reshape(1, 1)

    @pl.when(i == pl.num_programs(0) - 1)
    def _():
        out_ref[...] = jax.nn.sigmoid(out_ref[...] + bd_ref[...])


def _final_dot(flat, wd, bd):
    return pl.pallas_call(
        _final_body,
        grid=(N // BR,),
        in_specs=[pl.BlockSpec((BR, L), lambda i: (i, 0)),
                  pl.BlockSpec((BR, L), lambda i: (i, 0)),
                  pl.BlockSpec((1, 1), lambda i: (0, 0))],
        out_specs=pl.BlockSpec((1, 1), lambda i: (0, 0)),
        out_shape=jax.ShapeDtypeStruct((1, 1), jnp.float32),
    )(flat, wd, bd.reshape(1, 1))


def _gat1_shift_body(xl_ref, xr_ref, att_ref, o_ref):
    z = xl_ref[...] + xr_ref[...]
    lr = jnp.maximum(z, 0.2 * z) * att_ref[...]
    sh = jnp.sum(lr.reshape(lr.shape[0], 4, 256), axis=2)
    o_ref[...] = jnp.concatenate(
        [sh, jnp.zeros((sh.shape[0], 124), jnp.float32)], axis=1)


def _gat1_shift(xl, xr, attf):
    return pl.pallas_call(
        _gat1_shift_body,
        grid=(N // BR,),
        in_specs=[pl.BlockSpec((BR, 1024), lambda i: (i, 0)),
                  pl.BlockSpec((BR, 1024), lambda i: (i, 0)),
                  pl.BlockSpec((1, 1024), lambda i: (0, 0))],
        out_specs=pl.BlockSpec((BR, 128), lambda i: (i, 0)),
        out_shape=jax.ShapeDtypeStruct((N, 128), jnp.float32),
    )(xl, xr, attf.reshape(1, 1024))


def _gat2_shift_body(la_ref, ra_ref, att_ref, o_ref):
    z = la_ref[...] + ra_ref[...]
    o_ref[...] = jnp.maximum(z, 0.2 * z) * att_ref[...]


def _gat2_shift(xla, xra, att2):
    return pl.pallas_call(
        _gat2_shift_body,
        grid=(N // BR,),
        in_specs=[pl.BlockSpec((BR, 128), lambda i: (i, 0)),
                  pl.BlockSpec((BR, 128), lambda i: (i, 0)),
                  pl.BlockSpec((1, 1), lambda i: (0, 0))],
        out_specs=pl.BlockSpec((BR, 128), lambda i: (i, 0)),
        out_shape=jax.ShapeDtypeStruct((N, 128), jnp.float32),
    )(xla, xra, att2.reshape(1, 1))


def _mean1_body(p0_ref, p1_ref, o_ref):
    ps = p0_ref[...] + p1_ref[...]
    cnt = jnp.maximum(ps[:, 4:5], 1.0)
    m = ps[:, 0:4] / cnt
    o_ref[...] = jnp.concatenate(
        [m, jnp.zeros((m.shape[0], 124), jnp.float32)], axis=1)


def _mean1(p0, p1):
    return pl.pallas_call(
        _mean1_body,
        grid=(N // BR,),
        in_specs=[pl.BlockSpec((BR, 128), lambda i: (i, 0)),
                  pl.BlockSpec((BR, 128), lambda i: (i, 0))],
        out_specs=pl.BlockSpec((BR, 128), lambda i: (i, 0)),
        out_shape=jax.ShapeDtypeStruct((N, 128), jnp.float32),
    )(p0, p1)


def _gat1_norm_body(n_ref, d0_ref, d1_ref, b_ref, o_ref):
    den = (d0_ref[...] + d1_ref[...])[:, 0:4] + 1e-16
    scale = jnp.repeat(1.0 / den, 256, axis=1)
    o_ref[...] = n_ref[...] * scale + b_ref[...]


def _gat1_norm(num, d0, d1, bias1):
    return pl.pallas_call(
        _gat1_norm_body,
        grid=(N // BR,),
        in_specs=[pl.BlockSpec((BR, 1024), lambda i: (i, 0)),
                  pl.BlockSpec((BR, 128), lambda i: (i, 0)),
                  pl.BlockSpec((BR, 128), lambda i: (i, 0)),
                  pl.BlockSpec((1, 1024), lambda i: (0, 0))],
        out_specs=pl.BlockSpec((BR, 1024), lambda i: (i, 0)),
        out_shape=jax.ShapeDtypeStruct((N, 1024), jnp.float32),
    )(num, d0, d1, bias1.reshape(1, 1024))


def _gat2_proj_body(x_ref, wl_ref, bl_ref, wr_ref, br_ref, la_ref, ra_ref):
    x = x_ref[...]
    z = jnp.zeros((x.shape[0], 127), jnp.float32)
    xl2 = jnp.dot(x, wl_ref[...], preferred_element_type=jnp.float32) + bl_ref[...]
    xr2 = jnp.dot(x, wr_ref[...], preferred_element_type=jnp.float32) + br_ref[...]
    la_ref[...] = jnp.concatenate([xl2, z], axis=1)
    ra_ref[...] = jnp.concatenate([xr2, z], axis=1)


def _gat2_proj(r1, wl2, bl2, wr2, br2):
    return pl.pallas_call(
        _gat2_proj_body,
        grid=(N // BR,),
        in_specs=[pl.BlockSpec((BR, 1024), lambda i: (i, 0)),
                  pl.BlockSpec((1024, 1), lambda i: (0, 0)),
                  pl.BlockSpec((1, 1), lambda i: (0, 0)),
                  pl.BlockSpec((1024, 1), lambda i: (0, 0)),
                  pl.BlockSpec((1, 1), lambda i: (0, 0))],
        out_specs=[pl.BlockSpec((BR, 128), lambda i: (i, 0)),
                   pl.BlockSpec((BR, 128), lambda i: (i, 0))],
        out_shape=[jax.ShapeDtypeStruct((N, 128), jnp.float32),
                   jax.ShapeDtypeStruct((N, 128), jnp.float32)],
    )(r1, wl2, bl2.reshape(1, 1), wr2, br2.reshape(1, 1))


def _gat2_mean_body(p0_ref, p1_ref, o_ref):
    ps = p0_ref[...] + p1_ref[...]
    m = ps[:, 0:1] / jnp.maximum(ps[:, 1:2], 1.0)
    o_ref[...] = jnp.concatenate(
        [m, jnp.zeros((m.shape[0], 127), jnp.float32)], axis=1)


def _gat2_mean(p0, p1):
    return pl.pallas_call(
        _gat2_mean_body,
        grid=(N // BR,),
        in_specs=[pl.BlockSpec((BR, 128), lambda i: (i, 0)),
                  pl.BlockSpec((BR, 128), lambda i: (i, 0))],
        out_specs=pl.BlockSpec((BR, 128), lambda i: (i, 0)),
        out_shape=jax.ShapeDtypeStruct((N, 128), jnp.float32),
    )(p0, p1)


def _gat2_score_body(p0_ref, p1_ref, b_ref, o_ref):
    ps = p0_ref[...] + p1_ref[...]
    sc = ps[:, 0:1] / (ps[:, 1:2] + 1e-16) + b_ref[...]
    o_ref[...] = jnp.concatenate(
        [sc, jnp.zeros((sc.shape[0], 127), jnp.float32)], axis=1)


def _gat2_score(p0, p1, bias2):
    return pl.pallas_call(
        _gat2_score_body,
        grid=(N // BR,),
        in_specs=[pl.BlockSpec((BR, 128), lambda i: (i, 0)),
                  pl.BlockSpec((BR, 128), lambda i: (i, 0)),
                  pl.BlockSpec((1, 1), lambda i: (0, 0))],
        out_specs=pl.BlockSpec((BR, 128), lambda i: (i, 0)),
        out_shape=jax.ShapeDtypeStruct((N, 128), jnp.float32),
    )(p0, p1, bias2.reshape(1, 1))


def _gin_conv(x, agg, W1, b1, g, be, W2, b2, relu_out):
    h, s1, s2 = _mm_stats(x, agg, W1, b1)
    a, sh = _bn_scale(s1, s2, g, be)
    return _bn_mm(h, a, sh, W2, b2, relu_out)


def kernel(eeg_nodes, eeg_idx, W11, b11, g1, be1, W12, b12, W21, b21, g2, be2, W22, b22,
           Wl1, bl1, Wr1, br1, att1, bias1, Wl2, bl2, Wr2, br2, att2, bias2, Wd, bd):
    src = eeg_idx[0].astype(jnp.int32)
    dst = eeg_idx[1].astype(jnp.int32)
    E = src.shape[0]
    loop = jnp.arange(N, dtype=jnp.int32)
    E2 = E + N
    E2P = -(-E2 // (NW * 64)) * (NW * 64)
    s2p = jnp.concatenate([src, loop, jnp.zeros((E2P - E2,), jnp.int32)])
    d2p = jnp.concatenate([dst, loop,
                           jnp.full((E2P - E2,), SHR - 1, jnp.int32)])

    agg1 = _segsum(eeg_nodes, src, dst)
    h = _gin_conv(eeg_nodes, agg1, W11, b11, g1, be1, W12, b12, relu_out=True)
    agg2 = _segsum(h, src, dst)
    h = _gin_conv(h, agg2, W21, b21, g2, be2, W22, b22, relu_out=False)

    xl1, xr1 = _dual_mm(h, Wl1, bl1, Wr1, br1)
    xr1p = jnp.pad(xr1, ((0, SHR - N), (0, 0)))
    attf = att1.reshape(1024)
    meanp = jnp.pad(_gat1_shift(xl1, xr1, attf), ((0, SHR - N), (0, 0)))
    wp, wts = _gat1_wts(xl1, xr1p, s2p, d2p, attf, meanp, E2P)
    cols = []
    for j in range(8):
        pj = _wseg(xl1[:, j * 128:(j + 1) * 128], s2p, d2p, wts, j // 2)
        cols.append(_combine(pj[:N], pj[SHR:SHR + N]))
    num = jnp.concatenate(cols, axis=1)
    r1 = _gat1_norm(num, wp[:N], wp[SHR:SHR + N], bias1)

    xla, xra = _gat2_proj(r1, Wl2, bl2, Wr2, br2)
    xrap = jnp.pad(xra, ((0, SHR - N), (0, 0)))
    att2f = jnp.pad(att2.reshape(1), (0, 15))
    mean2p = jnp.pad(_gat2_shift(xla, xra, att2.reshape(1)),
                     ((0, SHR - N), (0, 0)))
    q = _gat2(xla, xrap, s2p, d2p, att2f, mean2p, mode="num")
    region_scores = _gat2_score(q[:N], q[SHR:SHR + N], bias2)[:, :1]

    dementia_pred = _final_dot(h, Wd.reshape(N, L), bd)
    return (dementia_pred, region_scores)


# speedup vs baseline: 5.4959x; 1.1611x over previous
import functools
import jax
import jax.numpy as jnp
from jax import lax
from jax.experimental import pallas as pl
from jax.experimental.pallas import tpu as pltpu
from jax.experimental.pallas import tpu_sc as plsc

N = 10000
T = 256
HID = 512
L = 256

BR = 1000

NW = 32
SHR = 10240
STRIPE = SHR // 16
BLK = 200

_mesh = plsc.VectorSubcoreMesh(core_axis_name="c", subcore_axis_name="s")


def _iota16():
    return lax.iota(jnp.int32, 16)


def _segsum_body(x_hbm, src_hbm, dst_hbm, outp_hbm,
                 idxbuf, dstbuf, rows, zbuf, shared, sem, sem2):
    E = src_hbm.shape[0]
    epw = E // NW
    c = lax.axis_index("c")
    s = lax.axis_index("s")
    w = s * 2 + c
    zbuf[...] = jnp.zeros_like(zbuf)
    for i in range(STRIPE // 64):
        pltpu.sync_copy(zbuf, shared.at[pl.ds(s * STRIPE + i * 64, 64)])
    plsc.subcore_barrier()
    base = w * epw

    def blk(b, _):
        off = pl.multiple_of(base + b * BLK, 8)
        c1 = pltpu.async_copy(src_hbm.at[pl.ds(off, BLK)], idxbuf, sem2)
        c2 = pltpu.async_copy(dst_hbm.at[pl.ds(off, BLK)], dstbuf, sem)
        c1.wait()
        c2.wait()
        pltpu.async_copy(x_hbm.at[idxbuf], rows, sem).wait()
        pltpu.async_copy(rows, shared.at[dstbuf], sem, add=True).wait()
        return 0

    lax.fori_loop(0, epw // BLK, blk, 0)
    plsc.subcore_barrier()
    pltpu.sync_copy(shared.at[pl.ds(s * STRIPE, STRIPE)],
                    outp_hbm.at[pl.ds(pl.multiple_of(c * SHR + s * STRIPE, 8),
                                      STRIPE)])


def _segsum128(x, src, dst):
    k = pl.kernel(
        _segsum_body,
        mesh=_mesh,
        out_type=jax.ShapeDtypeStruct((2 * SHR, 128), jnp.float32),
        scratch_types=[pltpu.VMEM((BLK,), jnp.int32),
                       pltpu.VMEM((BLK,), jnp.int32),
                       pltpu.VMEM((BLK, 128), jnp.float32),
                       pltpu.VMEM((64, 128), jnp.float32),
                       pltpu.VMEM_SHARED((SHR, 128), jnp.float32),
                       pltpu.SemaphoreType.DMA,
                       pltpu.SemaphoreType.DMA],
    )
    outp = k(x, src, dst)
    return outp[:N], outp[SHR:SHR + N]


def _combine_body(a_ref, b_ref, o_ref):
    o_ref[...] = a_ref[...] + b_ref[...]


def _combine(a, b):
    n, w_ = a.shape
    return pl.pallas_call(
        _combine_body,
        grid=(n // BR,),
        in_specs=[pl.BlockSpec((BR, w_), lambda i: (i, 0)),
                  pl.BlockSpec((BR, w_), lambda i: (i, 0))],
        out_specs=pl.BlockSpec((BR, w_), lambda i: (i, 0)),
        out_shape=jax.ShapeDtypeStruct((n, w_), jnp.float32),
    )(a, b)


def _segsum(x, src, dst):
    cols = []
    for j in range(x.shape[1] // 128):
        p0, p1 = _segsum128(x[:, j * 128:(j + 1) * 128], src, dst)
        cols.append(_combine(p0, p1))
    return jnp.concatenate(cols, axis=1) if len(cols) > 1 else cols[0]


GBLK = 64
GBLK1 = 16
GBLKW = 128


def _zero_shared(s, zbuf, shared):
    zbuf[...] = jnp.zeros_like(zbuf)
    for i in range(STRIPE // 64):
        pltpu.sync_copy(zbuf, shared.at[pl.ds(s * STRIPE + i * 64, 64)])
    plsc.subcore_barrier()


def _writeback(c, s, shared, outp_hbm):
    plsc.subcore_barrier()
    pltpu.sync_copy(shared.at[pl.ds(s * STRIPE, STRIPE)],
                    outp_hbm.at[pl.ds(pl.multiple_of(c * SHR + s * STRIPE, 8),
                                      STRIPE)])


def _zero_lrow_tail(lrow, nblk):
    def z(e, _):
        for cc in range(1, 8):
            lrow[e, pl.ds(cc * 16, 16)] = jnp.zeros((16,), jnp.float32)
        return 0

    lax.fori_loop(0, nblk, z, 0)


def _gat1_logits(rs, rd, attv, padf, e):
    louts = []
    for h in range(4):
        acc = jnp.zeros((16,), jnp.float32)
        for cc in range(16):
            o = h * 256 + cc * 16
            z = rs[e, pl.ds(o, 16)] + rd[e, pl.ds(o, 16)]
            lr = jnp.maximum(z, 0.2 * z)
            acc = acc + lr * attv[pl.ds(o, 16)]
        pre = acc
        for sh in (1, 2, 4, 8):
            padf[pl.ds(16, 16)] = pre
            pre = pre + padf[pl.ds(16 - sh, 16)]
        louts.append(pre[15])
    return louts


def _gat1_mean_body(xl_hbm, xrp_hbm, src_hbm, dst_hbm, att_hbm, outp_hbm,
                    sbuf, dbuf, rs, rd, attv, lrow, padf, zbuf, shared,
                    sem, sem2):
    E2P = src_hbm.shape[0]
    epw = E2P // NW
    c = lax.axis_index("c")
    s = lax.axis_index("s")
    w = s * 2 + c
    iota = _iota16()
    _zero_shared(s, zbuf, shared)
    _zero_lrow_tail(lrow, GBLK1)
    padf[pl.ds(0, 16)] = jnp.zeros((16,), jnp.float32)
    pltpu.sync_copy(att_hbm, attv)
    base = w * epw

    def blk(b, _):
        off = pl.multiple_of(base + b * GBLK1, 8)
        pltpu.sync_copy(src_hbm.at[pl.ds(off, GBLK1)], sbuf)
        pltpu.sync_copy(dst_hbm.at[pl.ds(off, GBLK1)], dbuf)
        c1 = pltpu.async_copy(xl_hbm.at[sbuf], rs, sem)
        c2 = pltpu.async_copy(xrp_hbm.at[dbuf], rd, sem2)
        c1.wait()
        c2.wait()

        def edge(e, _):
            louts = _gat1_logits(rs, rd, attv, padf, e)
            row = jnp.where(iota == 4, 1.0, 0.0).astype(jnp.float32)
            for h in range(4):
                row = jnp.where(iota == h, louts[h], row)
            lrow[e, pl.ds(0, 16)] = row
            return 0

        lax.fori_loop(0, GBLK1, edge, 0)
        pltpu.async_copy(lrow, shared.at[dbuf], sem, add=True).wait()
        return 0

    lax.fori_loop(0, epw // GBLK1, blk, 0)
    _writeback(c, s, shared, outp_hbm)


def _gat1_wts_body(xl_hbm, xrp_hbm, src_hbm, dst_hbm, att_hbm, meanp_hbm,
                   outp_hbm, w_hbm,
                   sbuf, dbuf, rs, rd, mr, attv, lrow, wstage, padf, zbuf,
                   shared, sem, sem2, sem3):
    E2P = src_hbm.shape[0]
    epw = E2P // NW
    c = lax.axis_index("c")
    s = lax.axis_index("s")
    w = s * 2 + c
    iota = _iota16()
    _zero_shared(s, zbuf, shared)
    _zero_lrow_tail(lrow, GBLK1)
    padf[pl.ds(0, 16)] = jnp.zeros((16,), jnp.float32)
    pltpu.sync_copy(att_hbm, attv)
    base = w * epw
    zf = jnp.zeros((16,), jnp.float32)

    def blk(b, _):
        off = pl.multiple_of(base + b * GBLK1, 8)
        pltpu.sync_copy(src_hbm.at[pl.ds(off, GBLK1)], sbuf)
        pltpu.sync_copy(dst_hbm.at[pl.ds(off, GBLK1)], dbuf)
        c1 = pltpu.async_copy(xl_hbm.at[sbuf], rs, sem)
        c2 = pltpu.async_copy(xrp_hbm.at[dbuf], rd, sem2)
        c3 = pltpu.async_copy(meanp_hbm.at[dbuf], mr, sem3)
        c1.wait()
        c2.wait()
        c3.wait()

        def edge(e, _):
            louts = _gat1_logits(rs, rd, attv, padf, e)
            mrow = mr[e, pl.ds(0, 16)]
            row = zf
            for h in range(4):
                wv = jnp.exp(jnp.minimum(zf + (louts[h] - mrow[h]), 75.0))
                row = jnp.where(iota == h, wv, row)
            lrow[e, pl.ds(0, 16)] = row
            wstage[pl.ds(e * 16, 16)] = row
            return 0

        lax.fori_loop(0, GBLK1, edge, 0)
        pltpu.async_copy(lrow, shared.at[dbuf], sem, add=True).wait()
        pltpu.sync_copy(wstage,
                        w_hbm.at[pl.ds(pl.multiple_of(off * 16, 8), GBLK1 * 16)])
        return 0

    lax.fori_loop(0, epw // GBLK1, blk, 0)
    _writeback(c, s, shared, outp_hbm)


def _wseg_body(xcol_hbm, src_hbm, dst_hbm, w_hbm, outp_hbm,
               sbuf, dbuf, wbuf, rows, zbuf, shared, sem, sem2, sem3,
               *, hlane):
    E2P = src_hbm.shape[0]
    epw = E2P // NW
    c = lax.axis_index("c")
    s = lax.axis_index("s")
    w = s * 2 + c
    _zero_shared(s, zbuf, shared)
    base = w * epw

    def blk(b, _):
        off = pl.multiple_of(base + b * GBLKW, 8)
        c1 = pltpu.async_copy(src_hbm.at[pl.ds(off, GBLKW)], sbuf, sem2)
        c2 = pltpu.async_copy(dst_hbm.at[pl.ds(off, GBLKW)], dbuf, sem3)
        c3 = pltpu.async_copy(
            w_hbm.at[pl.ds(pl.multiple_of(off * 16, 8), GBLKW * 16)], wbuf, sem)
        c1.wait()
        c2.wait()
        c3.wait()
        pltpu.async_copy(xcol_hbm.at[sbuf], rows, sem).wait()

        def edge(e, _):
            wv = wbuf[pl.ds(e * 16, 16)]
            ws = wv[hlane]
            for cc in range(8):
                rows[e, pl.ds(cc * 16, 16)] = rows[e, pl.ds(cc * 16, 16)] * ws
            return 0

        lax.fori_loop(0, GBLKW, edge, 0)
        pltpu.async_copy(rows, shared.at[dbuf], sem, add=True).wait()
        return 0

    lax.fori_loop(0, epw // GBLKW, blk, 0)
    _writeback(c, s, shared, outp_hbm)


def _gat2_body(xla_hbm, xrp_hbm, src_hbm, dst_hbm, att_hbm, meanp_hbm,
               outp_hbm, sbuf, dbuf, xa, xb, mr, attv, lrow, zbuf, shared,
               sem, sem2, sem3, *, mode):
    E2P = src_hbm.shape[0]
    epw = E2P // NW
    c = lax.axis_index("c")
    s = lax.axis_index("s")
    w = s * 2 + c
    iota = _iota16()
    _zero_shared(s, zbuf, shared)
    _zero_lrow_tail(lrow, GBLK)
    pltpu.sync_copy(att_hbm, attv)
    att2s = attv[...][0]
    base = w * epw
    zf = jnp.zeros((16,), jnp.float32)

    def blk(b, _):
        off = pl.multiple_of(base + b * GBLK, 8)
        pltpu.sync_copy(src_hbm.at[pl.ds(off, GBLK)], sbuf)
        pltpu.sync_copy(dst_hbm.at[pl.ds(off, GBLK)], dbuf)
        c1 = pltpu.async_copy(xla_hbm.at[sbuf], xa, sem)
        c2 = pltpu.async_copy(xrp_hbm.at[dbuf], xb, sem2)
        if mode == "num":
            pltpu.async_copy(meanp_hbm.at[dbuf], mr, sem3).wait()
        c1.wait()
        c2.wait()

        def edge(e, _):
            a0 = xa[e, pl.ds(0, 16)]
            b0 = xb[e, pl.ds(0, 16)]
            z = a0 + b0
            lr = jnp.maximum(z, 0.2 * z)
            lv = lr * att2s
            if mode == "mean":
                row = lv + jnp.where(iota == 1, 1.0, 0.0).astype(jnp.float32)
            else:
                m0 = mr[e, pl.ds(0, 16)]
                wv = jnp.exp(jnp.minimum(zf + (lv[0] - m0[0]), 75.0))
                row = jnp.where(iota == 0, wv * a0[0],
                                jnp.where(iota == 1, wv, zf)).astype(
                                    jnp.float32)
            lrow[e, pl.ds(0, 16)] = row
            return 0

        lax.fori_loop(0, GBLK, edge, 0)
        pltpu.async_copy(lrow, shared.at[dbuf], sem, add=True).wait()
        return 0

    lax.fori_loop(0, epw // GBLK, blk, 0)
    _writeback(c, s, shared, outp_hbm)


def _gat1_mean(xl, xrp, src, dst, attf):
    k = pl.kernel(
        _gat1_mean_body,
        mesh=_mesh,
        out_type=jax.ShapeDtypeStruct((2 * SHR, 128), jnp.float32),
        scratch_types=[pltpu.VMEM((GBLK1,), jnp.int32),
                       pltpu.VMEM((GBLK1,), jnp.int32),
                       pltpu.VMEM((GBLK1, 1024), jnp.float32),
                       pltpu.VMEM((GBLK1, 1024), jnp.float32),
                       pltpu.VMEM((1024,), jnp.float32),
                       pltpu.VMEM((GBLK1, 128), jnp.float32),
                       pltpu.VMEM((32,), jnp.float32),
                       pltpu.VMEM((64, 128), jnp.float32),
                       pltpu.VMEM_SHARED((SHR, 128), jnp.float32),
                       pltpu.SemaphoreType.DMA,
                       pltpu.SemaphoreType.DMA],
    )
    return k(xl, xrp, src, dst, attf)


def _gat1_wts(xl, xrp, src, dst, attf, meanp, e2p):
    k = pl.kernel(
        _gat1_wts_body,
        mesh=_mesh,
        out_type=[jax.ShapeDtypeStruct((2 * SHR, 128), jnp.float32),
                  jax.ShapeDtypeStruct((e2p * 16,), jnp.float32)],
        scratch_types=[pltpu.VMEM((GBLK1,), jnp.int32),
                       pltpu.VMEM((GBLK1,), jnp.int32),
                       pltpu.VMEM((GBLK1, 1024), jnp.float32),
                       pltpu.VMEM((GBLK1, 1024), jnp.float32),
                       pltpu.VMEM((GBLK1, 128), jnp.float32),
                       pltpu.VMEM((1024,), jnp.float32),
                       pltpu.VMEM((GBLK1, 128), jnp.float32),
                       pltpu.VMEM((GBLK1 * 16,), jnp.float32),
                       pltpu.VMEM((32,), jnp.float32),
                       pltpu.VMEM((64, 128), jnp.float32),
                       pltpu.VMEM_SHARED((SHR, 128), jnp.float32),
                       pltpu.SemaphoreType.DMA,
                       pltpu.SemaphoreType.DMA,
                       pltpu.SemaphoreType.DMA],
    )
    return k(xl, xrp, src, dst, attf, meanp)


def _wseg(xcol, src, dst, wts, hlane):
    k = pl.kernel(
        functools.partial(_wseg_body, hlane=hlane),
        mesh=_mesh,
        out_type=jax.ShapeDtypeStruct((2 * SHR, 128), jnp.float32),
        scratch_types=[pltpu.VMEM((GBLKW,), jnp.int32),
                       pltpu.VMEM((GBLKW,), jnp.int32),
                       pltpu.VMEM((GBLKW * 16,), jnp.float32),
                       pltpu.VMEM((GBLKW, 128), jnp.float32),
                       pltpu.VMEM((64, 128), jnp.float32),
                       pltpu.VMEM_SHARED((SHR, 128), jnp.float32),
                       pltpu.SemaphoreType.DMA,
                       pltpu.SemaphoreType.DMA,
                       pltpu.SemaphoreType.DMA],
    )
    return k(xcol, src, dst, wts)


def _gat2(xla, xrp, src, dst, att2f, meanp, mode):
    k = pl.kernel(
        functools.partial(_gat2_body, mode=mode),
        mesh=_mesh,
        out_type=jax.ShapeDtypeStruct((2 * SHR, 128), jnp.float32),
        scratch_types=[pltpu.VMEM((GBLK,), jnp.int32),
                       pltpu.VMEM((GBLK,), jnp.int32),
                       pltpu.VMEM((GBLK, 128), jnp.float32),
                       pltpu.VMEM((GBLK, 128), jnp.float32),
                       pltpu.VMEM((GBLK, 128), jnp.float32),
                       pltpu.VMEM((16,), jnp.float32),
                       pltpu.VMEM((GBLK, 128), jnp.float32),
                       pltpu.VMEM((64, 128), jnp.float32),
                       pltpu.VMEM_SHARED((SHR, 128), jnp.float32),
                       pltpu.SemaphoreType.DMA,
                       pltpu.SemaphoreType.DMA,
                       pltpu.SemaphoreType.DMA],
    )
    return k(xla, xrp, src, dst, att2f, meanp)


def _mm_stats_body(x_ref, agg_ref, w_ref, b_ref, h_ref, s1_ref, s2_ref):
    i = pl.program_id(0)
    u = x_ref[...] + agg_ref[...]
    h = jnp.dot(u, w_ref[...], preferred_element_type=jnp.float32) + b_ref[...]
    h_ref[...] = h

    @pl.when(i == 0)
    def _():
        s1_ref[...] = jnp.zeros_like(s1_ref)
        s2_ref[...] = jnp.zeros_like(s2_ref)

    s1_ref[...] += jnp.sum(h, axis=0, keepdims=True)
    s2_ref[...] += jnp.sum(h * h, axis=0, keepdims=True)


def _mm_stats(x, agg, w, b):
    n, k = x.shape
    c = w.shape[1]
    return pl.pallas_call(
        _mm_stats_body,
        grid=(n // BR,),
        in_specs=[
            pl.BlockSpec((BR, k), lambda i: (i, 0)),
            pl.BlockSpec((BR, k), lambda i: (i, 0)),
            pl.BlockSpec((k, c), lambda i: (0, 0)),
            pl.BlockSpec((1, c), lambda i: (0, 0)),
        ],
        out_specs=[
            pl.BlockSpec((BR, c), lambda i: (i, 0)),
            pl.BlockSpec((1, c), lambda i: (0, 0)),
            pl.BlockSpec((1, c), lambda i: (0, 0)),
        ],
        out_shape=[
            jax.ShapeDtypeStruct((n, c), jnp.float32),
            jax.ShapeDtypeStruct((1, c), jnp.float32),
            jax.ShapeDtypeStruct((1, c), jnp.float32),
        ],
    )(x, agg, w, b.reshape(1, c))


def _stats_body(s1_ref, s2_ref, g_ref, be_ref, a_ref, sh_ref):
    mean = s1_ref[...] * (1.0 / N)
    var = s2_ref[...] * (1.0 / N) - mean * mean
    a = g_ref[...] * jax.lax.rsqrt(var + 1e-5)
    a_ref[...] = a
    sh_ref[...] = be_ref[...] - mean * a


def _bn_scale(s1, s2, g, be):
    c = s1.shape[1]
    return pl.pallas_call(
        _stats_body,
        out_shape=[jax.ShapeDtypeStruct((1, c), jnp.float32),
                   jax.ShapeDtypeStruct((1, c), jnp.float32)],
    )(s1, s2, g.reshape(1, c), be.reshape(1, c))


def _bn_mm_body(h_ref, a_ref, sh_ref, w_ref, b_ref, o_ref, *, relu_out):
    t = jnp.maximum(h_ref[...] * a_ref[...] + sh_ref[...], 0.0)
    o = jnp.dot(t, w_ref[...], preferred_element_type=jnp.float32) + b_ref[...]
    if relu_out:
        o = jnp.maximum(o, 0.0)
    o_ref[...] = o


def _bn_mm(h, a, sh, w, b, relu_out):
    n, k = h.shape
    c = w.shape[1]
    return pl.pallas_call(
        functools.partial(_bn_mm_body, relu_out=relu_out),
        grid=(n // BR,),
        in_specs=[
            pl.BlockSpec((BR, k), lambda i: (i, 0)),
            pl.BlockSpec((1, k), lambda i: (0, 0)),
            pl.BlockSpec((1, k), lambda i: (0, 0)),
            pl.BlockSpec((k, c), lambda i: (0, 0)),
            pl.BlockSpec((1, c), lambda i: (0, 0)),
        ],
        out_specs=pl.BlockSpec((BR, c), lambda i: (i, 0)),
        out_shape=jax.ShapeDtypeStruct((n, c), jnp.float32),
    )(h, a, sh, w, b.reshape(1, c))


def _dual_mm_body(x_ref, wl_ref, bl_ref, wr_ref, br_ref, l_ref, r_ref):
    x = x_ref[...]
    l_ref[...] = jnp.dot(x, wl_ref[...], preferred_element_type=jnp.float32) + bl_ref[...]
    r_ref[...] = jnp.dot(x, wr_ref[...], preferred_element_type=jnp.float32) + br_ref[...]


def _dual_mm(x, wl, bl, wr, br):
    n, k = x.shape
    c = wl.shape[1]
    return pl.pallas_call(
        _dual_mm_body,
        grid=(n // BR,),
        in_specs=[
            pl.BlockSpec((BR, k), lambda i: (i, 0)),
            pl.BlockSpec((k, c), lambda i: (0, 0)),
            pl.BlockSpec((1, c), lambda i: (0, 0)),
            pl.BlockSpec((k, c), lambda i: (0, 0)),
            pl.BlockSpec((1, c), lambda i: (0, 0)),
        ],
        out_specs=[pl.BlockSpec((BR, c), lambda i: (i, 0)),
                   pl.BlockSpec((BR, c), lambda i: (i, 0))],
        out_shape=[jax.ShapeDtypeStruct((n, c), jnp.float32),
                   jax.ShapeDtypeStruct((n, c), jnp.float32)],
    )(x, wl, bl.reshape(1, c), wr, br.reshape(1, c))


def _final_body(flat_ref, wd_ref, bd_ref, out_ref):
    i = pl.program_id(0)

    @pl.when(i == 0)
    def _():
        out_ref[...] = jnp.zeros_like(out_ref)

    out_ref[...] += jnp.sum(flat_ref[...] * wd_ref[...]).reshape(1, 1)

    @pl.when(i == pl.num_programs(0) - 1)
    def _():
        out_ref[...] = jax.nn.sigmoid(out_ref[...] + bd_ref[...])


def _final_dot(flat, wd, bd):
    return pl.pallas_call(
        _final_body,
        grid=(N // BR,),
        in_specs=[pl.BlockSpec((BR, L), lambda i: (i, 0)),
                  pl.BlockSpec((BR, L), lambda i: (i, 0)),
                  pl.BlockSpec((1, 1), lambda i: (0, 0))],
        out_specs=pl.BlockSpec((1, 1), lambda i: (0, 0)),
        out_shape=jax.ShapeDtypeStruct((1, 1), jnp.float32),
    )(flat, wd, bd.reshape(1, 1))


def _gat1_shift_body(xl_ref, xr_ref, att_ref, o_ref):
    z = xl_ref[...] + xr_ref[...]
    lr = jnp.maximum(z, 0.2 * z) * att_ref[...]
    sh = jnp.sum(lr.reshape(lr.shape[0], 4, 256), axis=2)
    o_ref[...] = jnp.concatenate(
        [sh, jnp.zeros((sh.shape[0], 124), jnp.float32)], axis=1)


def _gat1_shift(xl, xr, attf):
    return pl.pallas_call(
        _gat1_shift_body,
        grid=(N // BR,),
        in_specs=[pl.BlockSpec((BR, 1024), lambda i: (i, 0)),
                  pl.BlockSpec((BR, 1024), lambda i: (i, 0)),
                  pl.BlockSpec((1, 1024), lambda i: (0, 0))],
        out_specs=pl.BlockSpec((BR, 128), lambda i: (i, 0)),
        out_shape=jax.ShapeDtypeStruct((N, 128), jnp.float32),
    )(xl, xr, attf.reshape(1, 1024))


def _gat2_shift_body(la_ref, ra_ref, att_ref, o_ref):
    z = la_ref[...] + ra_ref[...]
    o_ref[...] = jnp.maximum(z, 0.2 * z) * att_ref[...]


def _gat2_shift(xla, xra, att2):
    return pl.pallas_call(
        _gat2_shift_body,
        grid=(N // BR,),
        in_specs=[pl.BlockSpec((BR, 128), lambda i: (i, 0)),
                  pl.BlockSpec((BR, 128), lambda i: (i, 0)),
                  pl.BlockSpec((1, 1), lambda i: (0, 0))],
        out_specs=pl.BlockSpec((BR, 128), lambda i: (i, 0)),
        out_shape=jax.ShapeDtypeStruct((N, 128), jnp.float32),
    )(xla, xra, att2.reshape(1, 1))


def _mean1_body(p0_ref, p1_ref, o_ref):
    ps = p0_ref[...] + p1_ref[...]
    cnt = jnp.maximum(ps[:, 4:5], 1.0)
    m = ps[:, 0:4] / cnt
    o_ref[...] = jnp.concatenate(
        [m, jnp.zeros((m.shape[0], 124), jnp.float32)], axis=1)


def _mean1(p0, p1):
    return pl.pallas_call(
        _mean1_body,
        grid=(N // BR,),
        in_specs=[pl.BlockSpec((BR, 128), lambda i: (i, 0)),
                  pl.BlockSpec((BR, 128), lambda i: (i, 0))],
        out_specs=pl.BlockSpec((BR, 128), lambda i: (i, 0)),
        out_shape=jax.ShapeDtypeStruct((N, 128), jnp.float32),
    )(p0, p1)


def _gat1_norm_body(n_ref, d0_ref, d1_ref, b_ref, o_ref):
    den = (d0_ref[...] + d1_ref[...])[:, 0:4] + 1e-16
    scale = jnp.repeat(1.0 / den, 256, axis=1)
    o_ref[...] = n_ref[...] * scale + b_ref[...]


def _gat1_norm(num, d0, d1, bias1):
    return pl.pallas_call(
        _gat1_norm_body,
        grid=(N // BR,),
        in_specs=[pl.BlockSpec((BR, 1024), lambda i: (i, 0)),
                  pl.BlockSpec((BR, 128), lambda i: (i, 0)),
                  pl.BlockSpec((BR, 128), lambda i: (i, 0)),
                  pl.BlockSpec((1, 1024), lambda i: (0, 0))],
        out_specs=pl.BlockSpec((BR, 1024), lambda i: (i, 0)),
        out_shape=jax.ShapeDtypeStruct((N, 1024), jnp.float32),
    )(num, d0, d1, bias1.reshape(1, 1024))


def _gat2_proj_body(x_ref, wl_ref, bl_ref, wr_ref, br_ref, la_ref, ra_ref):
    x = x_ref[...]
    z = jnp.zeros((x.shape[0], 127), jnp.float32)
    xl2 = jnp.dot(x, wl_ref[...], preferred_element_type=jnp.float32) + bl_ref[...]
    xr2 = jnp.dot(x, wr_ref[...], preferred_element_type=jnp.float32) + br_ref[...]
    la_ref[...] = jnp.concatenate([xl2, z], axis=1)
    ra_ref[...] = jnp.concatenate([xr2, z], axis=1)


def _gat2_proj(r1, wl2, bl2, wr2, br2):
    return pl.pallas_call(
        _gat2_proj_body,
        grid=(N // BR,),
        in_specs=[pl.BlockSpec((BR, 1024), lambda i: (i, 0)),
                  pl.BlockSpec((1024, 1), lambda i: (0, 0)),
                  pl.BlockSpec((1, 1), lambda i: (0, 0)),
                  pl.BlockSpec((1024, 1), lambda i: (0, 0)),
                  pl.BlockSpec((1, 1), lambda i: (0, 0))],
        out_specs=[pl.BlockSpec((BR, 128), lambda i: (i, 0)),
                   pl.BlockSpec((BR, 128), lambda i: (i, 0))],
        out_shape=[jax.ShapeDtypeStruct((N, 128), jnp.float32),
                   jax.ShapeDtypeStruct((N, 128), jnp.float32)],
    )(r1, wl2, bl2.reshape(1, 1), wr2, br2.reshape(1, 1))


def _gat2_mean_body(p0_ref, p1_ref, o_ref):
    ps = p0_ref[...] + p1_ref[...]
    m = ps[:, 0:1] / jnp.maximum(ps[:, 1:2], 1.0)
    o_ref[...] = jnp.concatenate(
        [m, jnp.zeros((m.shape[0], 127), jnp.float32)], axis=1)


def _gat2_mean(p0, p1):
    return pl.pallas_call(
        _gat2_mean_body,
        grid=(N // BR,),
        in_specs=[pl.BlockSpec((BR, 128), lambda i: (i, 0)),
                  pl.BlockSpec((BR, 128), lambda i: (i, 0))],
        out_specs=pl.BlockSpec((BR, 128), lambda i: (i, 0)),
        out_shape=jax.ShapeDtypeStruct((N, 128), jnp.float32),
    )(p0, p1)


def _gat2_score_body(p0_ref, p1_ref, b_ref, o_ref):
    ps = p0_ref[...] + p1_ref[...]
    sc = ps[:, 0:1] / (ps[:, 1:2] + 1e-16) + b_ref[...]
    o_ref[...] = jnp.concatenate(
        [sc, jnp.zeros((sc.shape[0], 127), jnp.float32)], axis=1)


def _gat2_score(p0, p1, bias2):
    return pl.pallas_call(
        _gat2_score_body,
        grid=(N // BR,),
        in_specs=[pl.BlockSpec((BR, 128), lambda i: (i, 0)),
                  pl.BlockSpec((BR, 128), lambda i: (i, 0)),
                  pl.BlockSpec((1, 1), lambda i: (0, 0))],
        out_specs=pl.BlockSpec((BR, 128), lambda i: (i, 0)),
        out_shape=jax.ShapeDtypeStruct((N, 128), jnp.float32),
    )(p0, p1, bias2.reshape(1, 1))


def _gin_conv(x, agg, W1, b1, g, be, W2, b2, relu_out):
    h, s1, s2 = _mm_stats(x, agg, W1, b1)
    a, sh = _bn_scale(s1, s2, g, be)
    return _bn_mm(h, a, sh, W2, b2, relu_out)


def kernel(eeg_nodes, eeg_idx, W11, b11, g1, be1, W12, b12, W21, b21, g2, be2, W22, b22,
           Wl1, bl1, Wr1, br1, att1, bias1, Wl2, bl2, Wr2, br2, att2, bias2, Wd, bd):
    src = eeg_idx[0].astype(jnp.int32)
    dst = eeg_idx[1].astype(jnp.int32)
    E = src.shape[0]
    loop = jnp.arange(N, dtype=jnp.int32)
    E2 = E + N
    E2P = -(-E2 // (NW * 128)) * (NW * 128)
    s2p = jnp.concatenate([src, loop, jnp.zeros((E2P - E2,), jnp.int32)])
    d2p = jnp.concatenate([dst, loop,
                           jnp.full((E2P - E2,), SHR - 1, jnp.int32)])

    agg1 = _segsum(eeg_nodes, src, dst)
    h = _gin_conv(eeg_nodes, agg1, W11, b11, g1, be1, W12, b12, relu_out=True)
    agg2 = _segsum(h, src, dst)
    h = _gin_conv(h, agg2, W21, b21, g2, be2, W22, b22, relu_out=False)

    xl1, xr1 = _dual_mm(h, Wl1, bl1, Wr1, br1)
    xr1p = jnp.pad(xr1, ((0, SHR - N), (0, 0)))
    attf = att1.reshape(1024)
    meanp = jnp.pad(_gat1_shift(xl1, xr1, attf), ((0, SHR - N), (0, 0)))
    wp, wts = _gat1_wts(xl1, xr1p, s2p, d2p, attf, meanp, E2P)
    cols = []
    for j in range(8):
        pj = _wseg(xl1[:, j * 128:(j + 1) * 128], s2p, d2p, wts, j // 2)
        cols.append(_combine(pj[:N], pj[SHR:SHR + N]))
    num = jnp.concatenate(cols, axis=1)
    r1 = _gat1_norm(num, wp[:N], wp[SHR:SHR + N], bias1)

    xla, xra = _gat2_proj(r1, Wl2, bl2, Wr2, br2)
    xrap = jnp.pad(xra, ((0, SHR - N), (0, 0)))
    att2f = jnp.pad(att2.reshape(1), (0, 15))
    mean2p = jnp.pad(_gat2_shift(xla, xra, att2.reshape(1)),
                     ((0, SHR - N), (0, 0)))
    q = _gat2(xla, xrap, s2p, d2p, att2f, mean2p, mode="num")
    region_scores = _gat2_score(q[:N], q[SHR:SHR + N], bias2)[:, :1]

    dementia_pred = _final_dot(h, Wd.reshape(N, L), bd)
    return (dementia_pred, region_scores)


# speedup vs baseline: 5.6164x; 1.0219x over previous
import functools
import jax
import jax.numpy as jnp
from jax import lax
from jax.experimental import pallas as pl
from jax.experimental.pallas import tpu as pltpu
from jax.experimental.pallas import tpu_sc as plsc

N = 10000
T = 256
HID = 512
L = 256

BR = 1000

NW = 32
SHR = 10240
STRIPE = SHR // 16
BLK = 200

_mesh = plsc.VectorSubcoreMesh(core_axis_name="c", subcore_axis_name="s")


def _iota16():
    return lax.iota(jnp.int32, 16)


def _segsum_body(x_hbm, src_hbm, dst_hbm, outp_hbm,
                 idxbuf, dstbuf, rows, zbuf, shared, sem, sem2):
    E = src_hbm.shape[0]
    epw = E // NW
    c = lax.axis_index("c")
    s = lax.axis_index("s")
    w = s * 2 + c
    zbuf[...] = jnp.zeros_like(zbuf)
    for i in range(STRIPE // 64):
        pltpu.sync_copy(zbuf, shared.at[pl.ds(s * STRIPE + i * 64, 64)])
    plsc.subcore_barrier()
    base = w * epw

    def blk(b, _):
        off = pl.multiple_of(base + b * BLK, 8)
        c1 = pltpu.async_copy(src_hbm.at[pl.ds(off, BLK)], idxbuf, sem2)
        c2 = pltpu.async_copy(dst_hbm.at[pl.ds(off, BLK)], dstbuf, sem)
        c1.wait()
        c2.wait()
        pltpu.async_copy(x_hbm.at[idxbuf], rows, sem).wait()
        pltpu.async_copy(rows, shared.at[dstbuf], sem, add=True).wait()
        return 0

    lax.fori_loop(0, epw // BLK, blk, 0)
    plsc.subcore_barrier()
    pltpu.sync_copy(shared.at[pl.ds(s * STRIPE, STRIPE)],
                    outp_hbm.at[pl.ds(pl.multiple_of(c * SHR + s * STRIPE, 8),
                                      STRIPE)])


def _segsum128(x, src, dst):
    k = pl.kernel(
        _segsum_body,
        mesh=_mesh,
        out_type=jax.ShapeDtypeStruct((2 * SHR, 128), jnp.float32),
        scratch_types=[pltpu.VMEM((BLK,), jnp.int32),
                       pltpu.VMEM((BLK,), jnp.int32),
                       pltpu.VMEM((BLK, 128), jnp.float32),
                       pltpu.VMEM((64, 128), jnp.float32),
                       pltpu.VMEM_SHARED((SHR, 128), jnp.float32),
                       pltpu.SemaphoreType.DMA,
                       pltpu.SemaphoreType.DMA],
    )
    outp = k(x, src, dst)
    return outp[:N], outp[SHR:SHR + N]


def _combine_body(a_ref, b_ref, o_ref):
    o_ref[...] = a_ref[...] + b_ref[...]


def _combine(a, b):
    n, w_ = a.shape
    return pl.pallas_call(
        _combine_body,
        grid=(n // BR,),
        in_specs=[pl.BlockSpec((BR, w_), lambda i: (i, 0)),
                  pl.BlockSpec((BR, w_), lambda i: (i, 0))],
        out_specs=pl.BlockSpec((BR, w_), lambda i: (i, 0)),
        out_shape=jax.ShapeDtypeStruct((n, w_), jnp.float32),
    )(a, b)


def _segsum(x, src, dst):
    cols = []
    for j in range(x.shape[1] // 128):
        p0, p1 = _segsum128(x[:, j * 128:(j + 1) * 128], src, dst)
        cols.append(_combine(p0, p1))
    return jnp.concatenate(cols, axis=1) if len(cols) > 1 else cols[0]


GBLK = 64
GBLK1 = 16
GBLKW = 128


def _zero_shared(s, zbuf, shared):
    zbuf[...] = jnp.zeros_like(zbuf)
    for i in range(STRIPE // 64):
        pltpu.sync_copy(zbuf, shared.at[pl.ds(s * STRIPE + i * 64, 64)])
    plsc.subcore_barrier()


def _writeback(c, s, shared, outp_hbm):
    plsc.subcore_barrier()
    pltpu.sync_copy(shared.at[pl.ds(s * STRIPE, STRIPE)],
                    outp_hbm.at[pl.ds(pl.multiple_of(c * SHR + s * STRIPE, 8),
                                      STRIPE)])


def _zero_lrow_tail(lrow, nblk):
    def z(e, _):
        for cc in range(1, 8):
            lrow[e, pl.ds(cc * 16, 16)] = jnp.zeros((16,), jnp.float32)
        return 0

    lax.fori_loop(0, nblk, z, 0)


def _gat1_logits(rs, rd, attv, padf, e):
    louts = []
    for h in range(4):
        acc = jnp.zeros((16,), jnp.float32)
        for cc in range(16):
            o = h * 256 + cc * 16
            z = rs[e, pl.ds(o, 16)] + rd[e, pl.ds(o, 16)]
            lr = jnp.maximum(z, 0.2 * z)
            acc = acc + lr * attv[pl.ds(o, 16)]
        pre = acc
        for sh in (1, 2, 4, 8):
            padf[pl.ds(16, 16)] = pre
            pre = pre + padf[pl.ds(16 - sh, 16)]
        louts.append(pre[15])
    return louts


def _gat1_mean_body(xl_hbm, xrp_hbm, src_hbm, dst_hbm, att_hbm, outp_hbm,
                    sbuf, dbuf, rs, rd, attv, lrow, padf, zbuf, shared,
                    sem, sem2):
    E2P = src_hbm.shape[0]
    epw = E2P // NW
    c = lax.axis_index("c")
    s = lax.axis_index("s")
    w = s * 2 + c
    iota = _iota16()
    _zero_shared(s, zbuf, shared)
    _zero_lrow_tail(lrow, GBLK1)
    padf[pl.ds(0, 16)] = jnp.zeros((16,), jnp.float32)
    pltpu.sync_copy(att_hbm, attv)
    base = w * epw

    def blk(b, _):
        off = pl.multiple_of(base + b * GBLK1, 8)
        pltpu.sync_copy(src_hbm.at[pl.ds(off, GBLK1)], sbuf)
        pltpu.sync_copy(dst_hbm.at[pl.ds(off, GBLK1)], dbuf)
        c1 = pltpu.async_copy(xl_hbm.at[sbuf], rs, sem)
        c2 = pltpu.async_copy(xrp_hbm.at[dbuf], rd, sem2)
        c1.wait()
        c2.wait()

        def edge(e, _):
            louts = _gat1_logits(rs, rd, attv, padf, e)
            row = jnp.where(iota == 4, 1.0, 0.0).astype(jnp.float32)
            for h in range(4):
                row = jnp.where(iota == h, louts[h], row)
            lrow[e, pl.ds(0, 16)] = row
            return 0

        lax.fori_loop(0, GBLK1, edge, 0)
        pltpu.async_copy(lrow, shared.at[dbuf], sem, add=True).wait()
        return 0

    lax.fori_loop(0, epw // GBLK1, blk, 0)
    _writeback(c, s, shared, outp_hbm)


def _gat1_wts_body(xl_hbm, xrp_hbm, src_hbm, dst_hbm, att_hbm, meanp_hbm,
                   outp_hbm, w_hbm,
                   sbuf, dbuf, rs, rd, mr, attv, lrow, wstage, padf, zbuf,
                   shared, sem, sem2, sem3):
    E2P = src_hbm.shape[0]
    epw = E2P // NW
    c = lax.axis_index("c")
    s = lax.axis_index("s")
    w = s * 2 + c
    iota = _iota16()
    _zero_shared(s, zbuf, shared)
    _zero_lrow_tail(lrow, GBLK1)
    padf[pl.ds(0, 16)] = jnp.zeros((16,), jnp.float32)
    pltpu.sync_copy(att_hbm, attv)
    base = w * epw
    zf = jnp.zeros((16,), jnp.float32)

    def blk(b, _):
        off = pl.multiple_of(base + b * GBLK1, 8)
        i1 = pltpu.async_copy(src_hbm.at[pl.ds(off, GBLK1)], sbuf, sem)
        i2 = pltpu.async_copy(dst_hbm.at[pl.ds(off, GBLK1)], dbuf, sem2)
        i1.wait()
        i2.wait()
        c1 = pltpu.async_copy(xl_hbm.at[sbuf], rs, sem)
        c2 = pltpu.async_copy(xrp_hbm.at[dbuf], rd, sem2)
        c3 = pltpu.async_copy(meanp_hbm.at[dbuf], mr, sem3)
        c1.wait()
        c2.wait()
        c3.wait()

        def edge(e, _):
            louts = _gat1_logits(rs, rd, attv, padf, e)
            mrow = mr[e, pl.ds(0, 16)]
            row = zf
            for h in range(4):
                wv = jnp.exp(jnp.minimum(zf + (louts[h] - mrow[h]), 75.0))
                row = jnp.where(iota == h, wv, row)
            lrow[e, pl.ds(0, 16)] = row
            wstage[pl.ds(e * 16, 16)] = row
            return 0

        lax.fori_loop(0, GBLK1, edge, 0)
        pltpu.async_copy(lrow, shared.at[dbuf], sem, add=True).wait()
        pltpu.sync_copy(wstage,
                        w_hbm.at[pl.ds(pl.multiple_of(off * 16, 8), GBLK1 * 16)])
        return 0

    lax.fori_loop(0, epw // GBLK1, blk, 0)
    _writeback(c, s, shared, outp_hbm)


def _wseg_body(xcol_hbm, src_hbm, dst_hbm, w_hbm, outp_hbm,
               sbuf, dbuf, wbuf, rows, zbuf, shared, sem, sem2, sem3,
               *, hlane):
    E2P = src_hbm.shape[0]
    epw = E2P // NW
    c = lax.axis_index("c")
    s = lax.axis_index("s")
    w = s * 2 + c
    _zero_shared(s, zbuf, shared)
    base = w * epw

    def blk(b, _):
        off = pl.multiple_of(base + b * GBLKW, 8)
        c1 = pltpu.async_copy(src_hbm.at[pl.ds(off, GBLKW)], sbuf, sem2)
        c2 = pltpu.async_copy(dst_hbm.at[pl.ds(off, GBLKW)], dbuf, sem3)
        c3 = pltpu.async_copy(
            w_hbm.at[pl.ds(pl.multiple_of(off * 16, 8), GBLKW * 16)], wbuf, sem)
        c1.wait()
        c2.wait()
        c3.wait()
        pltpu.async_copy(xcol_hbm.at[sbuf], rows, sem).wait()

        def edge(e, _):
            wv = wbuf[pl.ds(e * 16, 16)]
            ws = wv[hlane]
            for cc in range(8):
                rows[e, pl.ds(cc * 16, 16)] = rows[e, pl.ds(cc * 16, 16)] * ws
            return 0

        lax.fori_loop(0, GBLKW, edge, 0)
        pltpu.async_copy(rows, shared.at[dbuf], sem, add=True).wait()
        return 0

    lax.fori_loop(0, epw // GBLKW, blk, 0)
    _writeback(c, s, shared, outp_hbm)


def _gat2_body(xla_hbm, xrp_hbm, src_hbm, dst_hbm, att_hbm, meanp_hbm,
               outp_hbm, sbuf, dbuf, xa, xb, mr, attv, lrow, zbuf, shared,
               sem, sem2, sem3, *, mode):
    E2P = src_hbm.shape[0]
    epw = E2P // NW
    c = lax.axis_index("c")
    s = lax.axis_index("s")
    w = s * 2 + c
    iota = _iota16()
    _zero_shared(s, zbuf, shared)
    _zero_lrow_tail(lrow, GBLK)
    pltpu.sync_copy(att_hbm, attv)
    att2s = attv[...][0]
    base = w * epw
    zf = jnp.zeros((16,), jnp.float32)

    def blk(b, _):
        off = pl.multiple_of(base + b * GBLK, 8)
        pltpu.sync_copy(src_hbm.at[pl.ds(off, GBLK)], sbuf)
        pltpu.sync_copy(dst_hbm.at[pl.ds(off, GBLK)], dbuf)
        c1 = pltpu.async_copy(xla_hbm.at[sbuf], xa, sem)
        c2 = pltpu.async_copy(xrp_hbm.at[dbuf], xb, sem2)
        if mode == "num":
            pltpu.async_copy(meanp_hbm.at[dbuf], mr, sem3).wait()
        c1.wait()
        c2.wait()

        def edge(e, _):
            a0 = xa[e, pl.ds(0, 16)]
            b0 = xb[e, pl.ds(0, 16)]
            z = a0 + b0
            lr = jnp.maximum(z, 0.2 * z)
            lv = lr * att2s
            if mode == "mean":
                row = lv + jnp.where(iota == 1, 1.0, 0.0).astype(jnp.float32)
            else:
                m0 = mr[e, pl.ds(0, 16)]
                wv = jnp.exp(jnp.minimum(zf + (lv[0] - m0[0]), 75.0))
                row = jnp.where(iota == 0, wv * a0[0],
                                jnp.where(iota == 1, wv, zf)).astype(
                                    jnp.float32)
            lrow[e, pl.ds(0, 16)] = row
            return 0

        lax.fori_loop(0, GBLK, edge, 0)
        pltpu.async_copy(lrow, shared.at[dbuf], sem, add=True).wait()
        return 0

    lax.fori_loop(0, epw // GBLK, blk, 0)
    _writeback(c, s, shared, outp_hbm)


def _gat1_mean(xl, xrp, src, dst, attf):
    k = pl.kernel(
        _gat1_mean_body,
        mesh=_mesh,
        out_type=jax.ShapeDtypeStruct((2 * SHR, 128), jnp.float32),
        scratch_types=[pltpu.VMEM((GBLK1,), jnp.int32),
                       pltpu.VMEM((GBLK1,), jnp.int32),
                       pltpu.VMEM((GBLK1, 1024), jnp.float32),
                       pltpu.VMEM((GBLK1, 1024), jnp.float32),
                       pltpu.VMEM((1024,), jnp.float32),
                       pltpu.VMEM((GBLK1, 128), jnp.float32),
                       pltpu.VMEM((32,), jnp.float32),
                       pltpu.VMEM((64, 128), jnp.float32),
                       pltpu.VMEM_SHARED((SHR, 128), jnp.float32),
                       pltpu.SemaphoreType.DMA,
                       pltpu.SemaphoreType.DMA],
    )
    return k(xl, xrp, src, dst, attf)


def _gat1_wts(xl, xrp, src, dst, attf, meanp, e2p):
    k = pl.kernel(
        _gat1_wts_body,
        mesh=_mesh,
        out_type=[jax.ShapeDtypeStruct((2 * SHR, 128), jnp.float32),
                  jax.ShapeDtypeStruct((e2p * 16,), jnp.float32)],
        scratch_types=[pltpu.VMEM((GBLK1,), jnp.int32),
                       pltpu.VMEM((GBLK1,), jnp.int32),
                       pltpu.VMEM((GBLK1, 1024), jnp.float32),
                       pltpu.VMEM((GBLK1, 1024), jnp.float32),
                       pltpu.VMEM((GBLK1, 128), jnp.float32),
                       pltpu.VMEM((1024,), jnp.float32),
                       pltpu.VMEM((GBLK1, 128), jnp.float32),
                       pltpu.VMEM((GBLK1 * 16,), jnp.float32),
                       pltpu.VMEM((32,), jnp.float32),
                       pltpu.VMEM((64, 128), jnp.float32),
                       pltpu.VMEM_SHARED((SHR, 128), jnp.float32),
                       pltpu.SemaphoreType.DMA,
                       pltpu.SemaphoreType.DMA,
                       pltpu.SemaphoreType.DMA],
    )
    return k(xl, xrp, src, dst, attf, meanp)


def _wseg(xcol, src, dst, wts, hlane):
    k = pl.kernel(
        functools.partial(_wseg_body, hlane=hlane),
        mesh=_mesh,
        out_type=jax.ShapeDtypeStruct((2 * SHR, 128), jnp.float32),
        scratch_types=[pltpu.VMEM((GBLKW,), jnp.int32),
                       pltpu.VMEM((GBLKW,), jnp.int32),
                       pltpu.VMEM((GBLKW * 16,), jnp.float32),
                       pltpu.VMEM((GBLKW, 128), jnp.float32),
                       pltpu.VMEM((64, 128), jnp.float32),
                       pltpu.VMEM_SHARED((SHR, 128), jnp.float32),
                       pltpu.SemaphoreType.DMA,
                       pltpu.SemaphoreType.DMA,
                       pltpu.SemaphoreType.DMA],
    )
    return k(xcol, src, dst, wts)


def _gat2(xla, xrp, src, dst, att2f, meanp, mode):
    k = pl.kernel(
        functools.partial(_gat2_body, mode=mode),
        mesh=_mesh,
        out_type=jax.ShapeDtypeStruct((2 * SHR, 128), jnp.float32),
        scratch_types=[pltpu.VMEM((GBLK,), jnp.int32),
                       pltpu.VMEM((GBLK,), jnp.int32),
                       pltpu.VMEM((GBLK, 128), jnp.float32),
                       pltpu.VMEM((GBLK, 128), jnp.float32),
                       pltpu.VMEM((GBLK, 128), jnp.float32),
                       pltpu.VMEM((16,), jnp.float32),
                       pltpu.VMEM((GBLK, 128), jnp.float32),
                       pltpu.VMEM((64, 128), jnp.float32),
                       pltpu.VMEM_SHARED((SHR, 128), jnp.float32),
                       pltpu.SemaphoreType.DMA,
                       pltpu.SemaphoreType.DMA,
                       pltpu.SemaphoreType.DMA],
    )
    return k(xla, xrp, src, dst, att2f, meanp)


def _mm_stats_body(x_ref, agg_ref, w_ref, b_ref, h_ref, s1_ref, s2_ref):
    i = pl.program_id(0)
    u = x_ref[...] + agg_ref[...]
    h = jnp.dot(u, w_ref[...], preferred_element_type=jnp.float32) + b_ref[...]
    h_ref[...] = h

    @pl.when(i == 0)
    def _():
        s1_ref[...] = jnp.zeros_like(s1_ref)
        s2_ref[...] = jnp.zeros_like(s2_ref)

    s1_ref[...] += jnp.sum(h, axis=0, keepdims=True)
    s2_ref[...] += jnp.sum(h * h, axis=0, keepdims=True)


def _mm_stats(x, agg, w, b):
    n, k = x.shape
    c = w.shape[1]
    return pl.pallas_call(
        _mm_stats_body,
        grid=(n // BR,),
        in_specs=[
            pl.BlockSpec((BR, k), lambda i: (i, 0)),
            pl.BlockSpec((BR, k), lambda i: (i, 0)),
            pl.BlockSpec((k, c), lambda i: (0, 0)),
            pl.BlockSpec((1, c), lambda i: (0, 0)),
        ],
        out_specs=[
            pl.BlockSpec((BR, c), lambda i: (i, 0)),
            pl.BlockSpec((1, c), lambda i: (0, 0)),
            pl.BlockSpec((1, c), lambda i: (0, 0)),
        ],
        out_shape=[
            jax.ShapeDtypeStruct((n, c), jnp.float32),
            jax.ShapeDtypeStruct((1, c), jnp.float32),
            jax.ShapeDtypeStruct((1, c), jnp.float32),
        ],
    )(x, agg, w, b.reshape(1, c))


def _stats_body(s1_ref, s2_ref, g_ref, be_ref, a_ref, sh_ref):
    mean = s1_ref[...] * (1.0 / N)
    var = s2_ref[...] * (1.0 / N) - mean * mean
    a = g_ref[...] * jax.lax.rsqrt(var + 1e-5)
    a_ref[...] = a
    sh_ref[...] = be_ref[...] - mean * a


def _bn_scale(s1, s2, g, be):
    c = s1.shape[1]
    return pl.pallas_call(
        _stats_body,
        out_shape=[jax.ShapeDtypeStruct((1, c), jnp.float32),
                   jax.ShapeDtypeStruct((1, c), jnp.float32)],
    )(s1, s2, g.reshape(1, c), be.reshape(1, c))


def _bn_mm_body(h_ref, a_ref, sh_ref, w_ref, b_ref, o_ref, *, relu_out):
    t = jnp.maximum(h_ref[...] * a_ref[...] + sh_ref[...], 0.0)
    o = jnp.dot(t, w_ref[...], preferred_element_type=jnp.float32) + b_ref[...]
    if relu_out:
        o = jnp.maximum(o, 0.0)
    o_ref[...] = o


def _bn_mm(h, a, sh, w, b, relu_out):
    n, k = h.shape
    c = w.shape[1]
    return pl.pallas_call(
        functools.partial(_bn_mm_body, relu_out=relu_out),
        grid=(n // BR,),
        in_specs=[
            pl.BlockSpec((BR, k), lambda i: (i, 0)),
            pl.BlockSpec((1, k), lambda i: (0, 0)),
            pl.BlockSpec((1, k), lambda i: (0, 0)),
            pl.BlockSpec((k, c), lambda i: (0, 0)),
            pl.BlockSpec((1, c), lambda i: (0, 0)),
        ],
        out_specs=pl.BlockSpec((BR, c), lambda i: (i, 0)),
        out_shape=jax.ShapeDtypeStruct((n, c), jnp.float32),
    )(h, a, sh, w, b.reshape(1, c))


def _dual_mm_body(x_ref, wl_ref, bl_ref, wr_ref, br_ref, l_ref, r_ref):
    x = x_ref[...]
    l_ref[...] = jnp.dot(x, wl_ref[...], preferred_element_type=jnp.float32) + bl_ref[...]
    r_ref[...] = jnp.dot(x, wr_ref[...], preferred_element_type=jnp.float32) + br_ref[...]


def _dual_mm(x, wl, bl, wr, br):
    n, k = x.shape
    c = wl.shape[1]
    return pl.pallas_call(
        _dual_mm_body,
        grid=(n // BR,),
        in_specs=[
            pl.BlockSpec((BR, k), lambda i: (i, 0)),
            pl.BlockSpec((k, c), lambda i: (0, 0)),
            pl.BlockSpec((1, c), lambda i: (0, 0)),
            pl.BlockSpec((k, c), lambda i: (0, 0)),
            pl.BlockSpec((1, c), lambda i: (0, 0)),
        ],
        out_specs=[pl.BlockSpec((BR, c), lambda i: (i, 0)),
                   pl.BlockSpec((BR, c), lambda i: (i, 0))],
        out_shape=[jax.ShapeDtypeStruct((n, c), jnp.float32),
                   jax.ShapeDtypeStruct((n, c), jnp.float32)],
    )(x, wl, bl.reshape(1, c), wr, br.reshape(1, c))


def _final_body(flat_ref, wd_ref, bd_ref, out_ref):
    i = pl.program_id(0)

    @pl.when(i == 0)
    def _():
        out_ref[...] = jnp.zeros_like(out_ref)

    out_ref[...] += jnp.sum(flat_ref[...] * wd_ref[...]).reshape(1, 1)

    @pl.when(i == pl.num_programs(0) - 1)
    def _():
        out_ref[...] = jax.nn.sigmoid(out_ref[...] + bd_ref[...])


def _final_dot(flat, wd, bd):
    return pl.pallas_call(
        _final_body,
        grid=(N // BR,),
        in_specs=[pl.BlockSpec((BR, L), lambda i: (i, 0)),
                  pl.BlockSpec((BR, L), lambda i: (i, 0)),
                  pl.BlockSpec((1, 1), lambda i: (0, 0))],
        out_specs=pl.BlockSpec((1, 1), lambda i: (0, 0)),
        out_shape=jax.ShapeDtypeStruct((1, 1), jnp.float32),
    )(flat, wd, bd.reshape(1, 1))


def _gat1_shift_body(xl_ref, xr_ref, att_ref, o_ref):
    z = xl_ref[...] + xr_ref[...]
    lr = jnp.maximum(z, 0.2 * z) * att_ref[...]
    sh = jnp.sum(lr.reshape(lr.shape[0], 4, 256), axis=2)
    o_ref[...] = jnp.concatenate(
        [sh, jnp.zeros((sh.shape[0], 124), jnp.float32)], axis=1)


def _gat1_shift(xl, xr, attf):
    return pl.pallas_call(
        _gat1_shift_body,
        grid=(N // BR,),
        in_specs=[pl.BlockSpec((BR, 1024), lambda i: (i, 0)),
                  pl.BlockSpec((BR, 1024), lambda i: (i, 0)),
                  pl.BlockSpec((1, 1024), lambda i: (0, 0))],
        out_specs=pl.BlockSpec((BR, 128), lambda i: (i, 0)),
        out_shape=jax.ShapeDtypeStruct((N, 128), jnp.float32),
    )(xl, xr, attf.reshape(1, 1024))


def _gat2_shift_body(la_ref, ra_ref, att_ref, o_ref):
    z = la_ref[...] + ra_ref[...]
    o_ref[...] = jnp.maximum(z, 0.2 * z) * att_ref[...]


def _gat2_shift(xla, xra, att2):
    return pl.pallas_call(
        _gat2_shift_body,
        grid=(N // BR,),
        in_specs=[pl.BlockSpec((BR, 128), lambda i: (i, 0)),
                  pl.BlockSpec((BR, 128), lambda i: (i, 0)),
                  pl.BlockSpec((1, 1), lambda i: (0, 0))],
        out_specs=pl.BlockSpec((BR, 128), lambda i: (i, 0)),
        out_shape=jax.ShapeDtypeStruct((N, 128), jnp.float32),
    )(xla, xra, att2.reshape(1, 1))


def _mean1_body(p0_ref, p1_ref, o_ref):
    ps = p0_ref[...] + p1_ref[...]
    cnt = jnp.maximum(ps[:, 4:5], 1.0)
    m = ps[:, 0:4] / cnt
    o_ref[...] = jnp.concatenate(
        [m, jnp.zeros((m.shape[0], 124), jnp.float32)], axis=1)


def _mean1(p0, p1):
    return pl.pallas_call(
        _mean1_body,
        grid=(N // BR,),
        in_specs=[pl.BlockSpec((BR, 128), lambda i: (i, 0)),
                  pl.BlockSpec((BR, 128), lambda i: (i, 0))],
        out_specs=pl.BlockSpec((BR, 128), lambda i: (i, 0)),
        out_shape=jax.ShapeDtypeStruct((N, 128), jnp.float32),
    )(p0, p1)


def _gat1_norm_body(n_ref, d0_ref, d1_ref, b_ref, o_ref):
    den = (d0_ref[...] + d1_ref[...])[:, 0:4] + 1e-16
    scale = jnp.repeat(1.0 / den, 256, axis=1)
    o_ref[...] = n_ref[...] * scale + b_ref[...]


def _gat1_norm(num, d0, d1, bias1):
    return pl.pallas_call(
        _gat1_norm_body,
        grid=(N // BR,),
        in_specs=[pl.BlockSpec((BR, 1024), lambda i: (i, 0)),
                  pl.BlockSpec((BR, 128), lambda i: (i, 0)),
                  pl.BlockSpec((BR, 128), lambda i: (i, 0)),
                  pl.BlockSpec((1, 1024), lambda i: (0, 0))],
        out_specs=pl.BlockSpec((BR, 1024), lambda i: (i, 0)),
        out_shape=jax.ShapeDtypeStruct((N, 1024), jnp.float32),
    )(num, d0, d1, bias1.reshape(1, 1024))


def _gat2_proj_body(x_ref, wl_ref, bl_ref, wr_ref, br_ref, la_ref, ra_ref):
    x = x_ref[...]
    z = jnp.zeros((x.shape[0], 127), jnp.float32)
    xl2 = jnp.dot(x, wl_ref[...], preferred_element_type=jnp.float32) + bl_ref[...]
    xr2 = jnp.dot(x, wr_ref[...], preferred_element_type=jnp.float32) + br_ref[...]
    la_ref[...] = jnp.concatenate([xl2, z], axis=1)
    ra_ref[...] = jnp.concatenate([xr2, z], axis=1)


def _gat2_proj(r1, wl2, bl2, wr2, br2):
    return pl.pallas_call(
        _gat2_proj_body,
        grid=(N // BR,),
        in_specs=[pl.BlockSpec((BR, 1024), lambda i: (i, 0)),
                  pl.BlockSpec((1024, 1), lambda i: (0, 0)),
                  pl.BlockSpec((1, 1), lambda i: (0, 0)),
                  pl.BlockSpec((1024, 1), lambda i: (0, 0)),
                  pl.BlockSpec((1, 1), lambda i: (0, 0))],
        out_specs=[pl.BlockSpec((BR, 128), lambda i: (i, 0)),
                   pl.BlockSpec((BR, 128), lambda i: (i, 0))],
        out_shape=[jax.ShapeDtypeStruct((N, 128), jnp.float32),
                   jax.ShapeDtypeStruct((N, 128), jnp.float32)],
    )(r1, wl2, bl2.reshape(1, 1), wr2, br2.reshape(1, 1))


def _gat2_mean_body(p0_ref, p1_ref, o_ref):
    ps = p0_ref[...] + p1_ref[...]
    m = ps[:, 0:1] / jnp.maximum(ps[:, 1:2], 1.0)
    o_ref[...] = jnp.concatenate(
        [m, jnp.zeros((m.shape[0], 127), jnp.float32)], axis=1)


def _gat2_mean(p0, p1):
    return pl.pallas_call(
        _gat2_mean_body,
        grid=(N // BR,),
        in_specs=[pl.BlockSpec((BR, 128), lambda i: (i, 0)),
                  pl.BlockSpec((BR, 128), lambda i: (i, 0))],
        out_specs=pl.BlockSpec((BR, 128), lambda i: (i, 0)),
        out_shape=jax.ShapeDtypeStruct((N, 128), jnp.float32),
    )(p0, p1)


def _gat2_score_body(p0_ref, p1_ref, b_ref, o_ref):
    ps = p0_ref[...] + p1_ref[...]
    sc = ps[:, 0:1] / (ps[:, 1:2] + 1e-16) + b_ref[...]
    o_ref[...] = jnp.concatenate(
        [sc, jnp.zeros((sc.shape[0], 127), jnp.float32)], axis=1)


def _gat2_score(p0, p1, bias2):
    return pl.pallas_call(
        _gat2_score_body,
        grid=(N // BR,),
        in_specs=[pl.BlockSpec((BR, 128), lambda i: (i, 0)),
                  pl.BlockSpec((BR, 128), lambda i: (i, 0)),
                  pl.BlockSpec((1, 1), lambda i: (0, 0))],
        out_specs=pl.BlockSpec((BR, 128), lambda i: (i, 0)),
        out_shape=jax.ShapeDtypeStruct((N, 128), jnp.float32),
    )(p0, p1, bias2.reshape(1, 1))


def _gin_conv(x, agg, W1, b1, g, be, W2, b2, relu_out):
    h, s1, s2 = _mm_stats(x, agg, W1, b1)
    a, sh = _bn_scale(s1, s2, g, be)
    return _bn_mm(h, a, sh, W2, b2, relu_out)


def kernel(eeg_nodes, eeg_idx, W11, b11, g1, be1, W12, b12, W21, b21, g2, be2, W22, b22,
           Wl1, bl1, Wr1, br1, att1, bias1, Wl2, bl2, Wr2, br2, att2, bias2, Wd, bd):
    src = eeg_idx[0].astype(jnp.int32)
    dst = eeg_idx[1].astype(jnp.int32)
    E = src.shape[0]
    loop = jnp.arange(N, dtype=jnp.int32)
    E2 = E + N
    E2P = -(-E2 // (NW * 128)) * (NW * 128)
    s2p = jnp.concatenate([src, loop, jnp.zeros((E2P - E2,), jnp.int32)])
    d2p = jnp.concatenate([dst, loop,
                           jnp.full((E2P - E2,), SHR - 1, jnp.int32)])

    agg1 = _segsum(eeg_nodes, src, dst)
    h = _gin_conv(eeg_nodes, agg1, W11, b11, g1, be1, W12, b12, relu_out=True)
    agg2 = _segsum(h, src, dst)
    h = _gin_conv(h, agg2, W21, b21, g2, be2, W22, b22, relu_out=False)

    xl1, xr1 = _dual_mm(h, Wl1, bl1, Wr1, br1)
    xr1p = jnp.pad(xr1, ((0, SHR - N), (0, 0)))
    attf = att1.reshape(1024)
    meanp = jnp.pad(_gat1_shift(xl1, xr1, attf), ((0, SHR - N), (0, 0)))
    wp, wts = _gat1_wts(xl1, xr1p, s2p, d2p, attf, meanp, E2P)
    cols = []
    for j in range(8):
        pj = _wseg(xl1[:, j * 128:(j + 1) * 128], s2p, d2p, wts, j // 2)
        cols.append(_combine(pj[:N], pj[SHR:SHR + N]))
    num = jnp.concatenate(cols, axis=1)
    r1 = _gat1_norm(num, wp[:N], wp[SHR:SHR + N], bias1)

    xla, xra = _gat2_proj(r1, Wl2, bl2, Wr2, br2)
    xrap = jnp.pad(xra, ((0, SHR - N), (0, 0)))
    att2f = jnp.pad(att2.reshape(1), (0, 15))
    mean2p = jnp.pad(_gat2_shift(xla, xra, att2.reshape(1)),
                     ((0, SHR - N), (0, 0)))
    q = _gat2(xla, xrap, s2p, d2p, att2f, mean2p, mode="num")
    region_scores = _gat2_score(q[:N], q[SHR:SHR + N], bias2)[:, :1]

    dementia_pred = _final_dot(h, Wd.reshape(N, L), bd)
    return (dementia_pred, region_scores)
